# Initial kernel scaffold; baseline (speedup 1.0000x reference)
import jax
import jax.numpy as jnp
from jax import lax
from jax.experimental import pallas as pl
from jax.experimental.pallas import tpu as pltpu
from jax.experimental.pallas import tpu_sc as plsc

N = 100000
NPAD = 102400
E = 1600000
EPAD = 1605632
EROWS = 12544
CH = 32
HALF = 16
ENC = 128
NG = 256
NGPAD = 264
NC = 2
NS = 16
TILE_NODES = NPAD // NS
LROWS = EROWS // NS
DROWS = EROWS // (NC * NS)
GRP = 8

_mesh = plsc.VectorSubcoreMesh(core_axis_name="c", subcore_axis_name="s",
                               num_cores=NC, num_subcores=NS)


def _deg_body(dst_hbm, z1_hbm, ones_hbm, deg_hbm, didx_v, ones_v, deg_sh):
    c = lax.axis_index("c")
    t = lax.axis_index("s")
    pltpu.sync_copy(z1_hbm, deg_sh.at[pl.ds(t * TILE_NODES, TILE_NODES)])
    pltpu.sync_copy(ones_hbm, ones_v)
    plsc.subcore_barrier()
    base = (c * NS + t) * DROWS

    def grp(g, _):
        row0 = base + g * GRP
        pltpu.sync_copy(dst_hbm.at[pl.ds(row0, GRP)], didx_v)
        for j in range(GRP):
            pltpu.sync_copy(ones_v, deg_sh.at[didx_v.at[j]], add=True)
        return 0

    lax.fori_loop(0, DROWS // GRP, grp, 0)
    plsc.subcore_barrier()
    pltpu.sync_copy(deg_sh.at[pl.ds(t * TILE_NODES, TILE_NODES)],
                    deg_hbm.at[c, pl.ds(t * TILE_NODES, TILE_NODES)])


def _sc_degree(dst2d, z1, ones):
    return pl.kernel(
        _deg_body,
        out_type=jax.ShapeDtypeStruct((NC, NPAD), jnp.float32),
        mesh=_mesh,
        scratch_types=[
            pltpu.VMEM((GRP, 128), jnp.int32),
            pltpu.VMEM((128,), jnp.float32),
            pltpu.VMEM_SHARED((NPAD,), jnp.float32),
        ],
    )(dst2d, z1, ones)


def _layer_body(u_hbm, src2_hbm, dst_hbm, z16_hbm, s_hbm,
                idx_v, didx_v, rows_v, agg_sh, gsem):
    c = lax.axis_index("c")
    t = lax.axis_index("s")
    pltpu.sync_copy(z16_hbm, agg_sh.at[pl.ds(t * TILE_NODES, TILE_NODES)])
    plsc.subcore_barrier()
    base = t * LROWS

    def grp(g, _):
        row0 = base + g * GRP
        pltpu.sync_copy(src2_hbm.at[c, pl.ds(row0, GRP)], idx_v)
        pltpu.sync_copy(dst_hbm.at[pl.ds(row0, GRP)], didx_v)
        cps = [pltpu.async_copy(u_hbm.at[idx_v.at[j]], rows_v.at[j], gsem)
               for j in range(GRP)]
        for cp in cps:
            cp.wait()
        for j in range(GRP):
            pltpu.sync_copy(rows_v.at[j], agg_sh.at[didx_v.at[j]], add=True)
        return 0

    lax.fori_loop(0, LROWS // GRP, grp, 0)
    plsc.subcore_barrier()
    pltpu.sync_copy(agg_sh.at[pl.ds(t * TILE_NODES, TILE_NODES)],
                    s_hbm.at[c, pl.ds(t * TILE_NODES, TILE_NODES)])


def _sc_layer(uflat, src2, dst2d, z16):
    return pl.kernel(
        _layer_body,
        out_type=jax.ShapeDtypeStruct((NC, NPAD, HALF), jnp.float32),
        mesh=_mesh,
        scratch_types=[
            pltpu.VMEM((GRP, 128), jnp.int32),
            pltpu.VMEM((GRP, 128), jnp.int32),
            pltpu.VMEM((GRP, 128, HALF), jnp.float32),
            pltpu.VMEM_SHARED((NPAD, HALF), jnp.float32),
            pltpu.SemaphoreType.DMA,
        ],
    )(uflat, src2, dst2d, z16)


def _pool_body(h_hbm, batch_hbm, zp_hbm, out_hbm, bidx_v, rows_v, pool_sh):
    c = lax.axis_index("c")
    t = lax.axis_index("s")

    @pl.when(t == 0)
    def _():
        pltpu.sync_copy(zp_hbm, pool_sh)

    plsc.subcore_barrier()
    base = t * TILE_NODES

    def chunk(k, _):
        off = base + k * 128
        pltpu.sync_copy(batch_hbm.at[pl.ds(off, 128)], bidx_v.at[0])
        pltpu.sync_copy(h_hbm.at[c, pl.ds(off, 128)], rows_v)
        pltpu.sync_copy(rows_v, pool_sh.at[bidx_v.at[0]], add=True)
        return 0

    lax.fori_loop(0, TILE_NODES // 128, chunk, 0)
    plsc.subcore_barrier()

    @pl.when(t == 0)
    def _():
        pltpu.sync_copy(pool_sh, out_hbm.at[c])


def _sc_pool(h2, batch_pad, zpool):
    return pl.kernel(
        _pool_body,
        out_type=jax.ShapeDtypeStruct((NC, NGPAD, HALF), jnp.float32),
        mesh=_mesh,
        scratch_types=[
            pltpu.VMEM((1, 128), jnp.int32),
            pltpu.VMEM((128, HALF), jnp.float32),
            pltpu.VMEM_SHARED((NGPAD, HALF), jnp.float32),
        ],
    )(h2, batch_pad, zpool)


def _prep_kernel(src_ref, out_ref):
    s = src_ref[...]
    out_ref[...] = jnp.stack([s, s + NPAD], axis=0)


def _tc_prep(src2d):
    return pl.pallas_call(
        _prep_kernel,
        grid=(128,),
        in_specs=[pl.BlockSpec((98, 128), lambda i: (i, 0))],
        out_specs=pl.BlockSpec((2, 98, 128), lambda i: (0, i, 0)),
        out_shape=jax.ShapeDtypeStruct((2, EROWS, 128), jnp.int32),
    )(src2d)


def _init_kernel(deg_ref, x_ref, emb_ref, pw0_ref, pb0_ref, pw1_ref, pb1_ref,
                 cw0_ref, dinv_ref, u0_ref):
    deg = deg_ref[0] + deg_ref[1] + 1.0
    dinv = lax.rsqrt(jnp.maximum(deg, 1.0))
    dinv_ref[...] = dinv
    tab = jnp.maximum(emb_ref[...] @ pw0_ref[...] + pb0_ref[...], 0.0)
    tab = jnp.maximum(tab @ pw1_ref[...] + pb1_ref[...], 0.0)
    tab = tab @ cw0_ref[...]
    onehot = (x_ref[...] == lax.broadcasted_iota(jnp.int32, (1, ENC), 1))
    u0 = jnp.dot(onehot.astype(jnp.float32), tab,
                 preferred_element_type=jnp.float32) * dinv
    u0_ref[...] = jnp.stack([u0[:, :HALF], u0[:, HALF:]], axis=0)


def _tc_init(deg, x_pad, emb, pw0, pb0, pw1, pb1, cw0):
    R = NPAD // 32
    return pl.pallas_call(
        _init_kernel,
        grid=(32,),
        in_specs=[
            pl.BlockSpec((2, R, 1), lambda i: (0, i, 0)),
            pl.BlockSpec((R, 1), lambda i: (i, 0)),
            pl.BlockSpec((ENC, CH), lambda i: (0, 0)),
            pl.BlockSpec((CH, CH), lambda i: (0, 0)),
            pl.BlockSpec((1, CH), lambda i: (0, 0)),
            pl.BlockSpec((CH, CH), lambda i: (0, 0)),
            pl.BlockSpec((1, CH), lambda i: (0, 0)),
            pl.BlockSpec((CH, CH), lambda i: (0, 0)),
        ],
        out_specs=[
            pl.BlockSpec((R, 1), lambda i: (i, 0)),
            pl.BlockSpec((2, R, HALF), lambda i: (0, i, 0)),
        ],
        out_shape=[
            jax.ShapeDtypeStruct((NPAD, 1), jnp.float32),
            jax.ShapeDtypeStruct((2, NPAD, HALF), jnp.float32),
        ],
    )(deg, x_pad, emb, pw0, pb0, pw1, pb1, cw0)


def _mid_kernel(s_ref, u_ref, dinv_ref, b_ref, w_ref, out_ref):
    acc = s_ref[...] + u_ref[...]
    h = jnp.concatenate([acc[0], acc[1]], axis=-1)
    h = jnp.maximum(h * dinv_ref[...] + b_ref[...], 0.0)
    un = jnp.dot(h, w_ref[...], preferred_element_type=jnp.float32)
    un = un * dinv_ref[...]
    out_ref[...] = jnp.stack([un[:, :HALF], un[:, HALF:]], axis=0)


def _last_kernel(s_ref, u_ref, dinv_ref, b_ref, out_ref):
    acc = s_ref[...] + u_ref[...]
    h = jnp.concatenate([acc[0], acc[1]], axis=-1)
    h = jnp.maximum(h * dinv_ref[...] + b_ref[...], 0.0)
    out_ref[...] = jnp.stack([h[:, :HALF], h[:, HALF:]], axis=0)


def _tc_mid(s, u, dinv, b, w):
    R = NPAD // 32
    return pl.pallas_call(
        _mid_kernel,
        grid=(32,),
        in_specs=[
            pl.BlockSpec((2, R, HALF), lambda i: (0, i, 0)),
            pl.BlockSpec((2, R, HALF), lambda i: (0, i, 0)),
            pl.BlockSpec((R, 1), lambda i: (i, 0)),
            pl.BlockSpec((1, CH), lambda i: (0, 0)),
            pl.BlockSpec((CH, CH), lambda i: (0, 0)),
        ],
        out_specs=pl.BlockSpec((2, R, HALF), lambda i: (0, i, 0)),
        out_shape=jax.ShapeDtypeStruct((2, NPAD, HALF), jnp.float32),
    )(s, u, dinv, b, w)


def _tc_last(s, u, dinv, b):
    R = NPAD // 32
    return pl.pallas_call(
        _last_kernel,
        grid=(32,),
        in_specs=[
            pl.BlockSpec((2, R, HALF), lambda i: (0, i, 0)),
            pl.BlockSpec((2, R, HALF), lambda i: (0, i, 0)),
            pl.BlockSpec((R, 1), lambda i: (i, 0)),
            pl.BlockSpec((1, CH), lambda i: (0, 0)),
        ],
        out_specs=pl.BlockSpec((2, R, HALF), lambda i: (0, i, 0)),
        out_shape=jax.ShapeDtypeStruct((2, NPAD, HALF), jnp.float32),
    )(s, u, dinv, b)


def _post_kernel(p_ref, w0_ref, b0_ref, w1_ref, b1_ref, pw_ref, pb_ref,
                 out_ref):
    g = jnp.concatenate([p_ref[0, :NG], p_ref[1, :NG]], axis=-1)
    g = jnp.maximum(g @ w0_ref[...] + b0_ref[...], 0.0)
    g = jnp.maximum(g @ w1_ref[...] + b1_ref[...], 0.0)
    props = g @ pw_ref[...] + pb_ref[...]
    out_ref[...] = props[:, 0:1]


def _tc_post(pooled, w0, b0, w1, b1, pw, pb):
    return pl.pallas_call(
        _post_kernel,
        out_shape=jax.ShapeDtypeStruct((NG, 1), jnp.float32),
    )(pooled, w0, b0, w1, b1, pw, pb)


def kernel(x, edge_index, batch, emb, pre_W, pre_b, conv_W, conv_b,
           post_W, post_b, prop_W, prop_b):
    f32 = jnp.float32
    x_pad = jnp.concatenate([x, jnp.zeros((NPAD - N,), jnp.int32)])
    x_pad = x_pad.reshape(NPAD, 1)
    batch_pad = jnp.concatenate(
        [batch, jnp.full((NPAD - N,), NG, jnp.int32)])
    pad_e = EPAD - E
    src_pad = jnp.concatenate(
        [edge_index[0],
         (jnp.arange(pad_e, dtype=jnp.int32) * 997) % N])
    dst_pad = jnp.concatenate(
        [edge_index[1], jnp.full((pad_e,), N, jnp.int32)])
    src2d = src_pad.reshape(EROWS, 128)
    dst2d = dst_pad.reshape(EROWS, 128)
    z1 = jnp.zeros((TILE_NODES,), f32)
    ones = jnp.ones((128,), f32)
    z16 = jnp.zeros((TILE_NODES, HALF), f32)
    zpool = jnp.zeros((NGPAD, HALF), f32)
    b2 = [b.reshape(1, CH) for b in conv_b]
    pb2 = [b.reshape(1, CH) for b in pre_b]

    src2 = _tc_prep(src2d)
    deg = _sc_degree(dst2d, z1, ones)
    dinv, u = _tc_init(deg.reshape(2, NPAD, 1), x_pad, emb,
                       pre_W[0], pb2[0], pre_W[1], pb2[1], conv_W[0])
    h2 = None
    for l in range(6):
        s = _sc_layer(u.reshape(2 * NPAD, HALF), src2, dst2d, z16)
        if l < 5:
            u = _tc_mid(s, u, dinv, b2[l], conv_W[l + 1])
        else:
            h2 = _tc_last(s, u, dinv, b2[l])
    pooled = _sc_pool(h2, batch_pad, zpool)
    out = _tc_post(pooled, post_W[0], post_b[0].reshape(1, -1),
                   post_W[1], post_b[1].reshape(1, -1),
                   prop_W, prop_b.reshape(1, -1))
    return out


# speedup vs baseline: 17.5846x; 17.5846x over previous
import jax
import jax.numpy as jnp
from jax import lax
from jax.experimental import pallas as pl
from jax.experimental.pallas import tpu as pltpu
from jax.experimental.pallas import tpu_sc as plsc

N = 100000
NPAD = 102400
E = 1600000
EPAD = 1605632
EROWS = 12544
CH = 32
HALF = 16
ENC = 128
NG = 256
NGPAD = 264
NC = 2
NS = 16
TILE_NODES = NPAD // NS
LROWS = EROWS // NS
DROWS = EROWS // (NC * NS)
GRP = 8

_mesh = plsc.VectorSubcoreMesh(core_axis_name="c", subcore_axis_name="s",
                               num_cores=NC, num_subcores=NS)
_sc_params = pltpu.CompilerParams(use_tc_tiling_on_sc=False)


def _deg_body(dst_hbm, z1_hbm, ones_hbm, deg_hbm, didx_v, ones_v, deg_sh):
    c = lax.axis_index("c")
    t = lax.axis_index("s")
    pltpu.sync_copy(z1_hbm, deg_sh.at[pl.ds(t * TILE_NODES, TILE_NODES)])
    pltpu.sync_copy(ones_hbm, ones_v)
    plsc.subcore_barrier()
    base = (c * NS + t) * DROWS

    def grp(g, _):
        row0 = base + g * GRP
        pltpu.sync_copy(dst_hbm.at[pl.ds(row0, GRP)], didx_v)
        for j in range(GRP):
            pltpu.sync_copy(ones_v, deg_sh.at[didx_v.at[j]], add=True)
        return 0

    lax.fori_loop(0, DROWS // GRP, grp, 0)
    plsc.subcore_barrier()
    pltpu.sync_copy(deg_sh.at[pl.ds(t * TILE_NODES, TILE_NODES)],
                    deg_hbm.at[c, pl.ds(t * TILE_NODES, TILE_NODES)])


def _sc_degree(dst2d, z1, ones):
    return pl.kernel(
        _deg_body,
        out_type=jax.ShapeDtypeStruct((NC, NPAD), jnp.float32),
        mesh=_mesh,
        compiler_params=_sc_params,
        scratch_types=[
            pltpu.VMEM((GRP, 128), jnp.int32),
            pltpu.VMEM((128,), jnp.float32),
            pltpu.VMEM_SHARED((NPAD,), jnp.float32),
        ],
    )(dst2d, z1, ones)


def _layer_body(u_hbm, src2_hbm, dst_hbm, z16_hbm, s_hbm,
                idx_v, didx_v, rows_v, agg_sh, gsem):
    c = lax.axis_index("c")
    t = lax.axis_index("s")
    pltpu.sync_copy(z16_hbm, agg_sh.at[pl.ds(t * TILE_NODES, TILE_NODES)])
    plsc.subcore_barrier()
    base = t * LROWS

    def grp(g, _):
        row0 = base + g * GRP
        pltpu.sync_copy(src2_hbm.at[c, pl.ds(row0, GRP)], idx_v)
        pltpu.sync_copy(dst_hbm.at[pl.ds(row0, GRP)], didx_v)
        cps = [pltpu.async_copy(u_hbm.at[idx_v.at[j]], rows_v.at[j], gsem)
               for j in range(GRP)]
        for cp in cps:
            cp.wait()
        for j in range(GRP):
            pltpu.sync_copy(rows_v.at[j], agg_sh.at[didx_v.at[j]], add=True)
        return 0

    lax.fori_loop(0, LROWS // GRP, grp, 0)
    plsc.subcore_barrier()
    pltpu.sync_copy(agg_sh.at[pl.ds(t * TILE_NODES, TILE_NODES)],
                    s_hbm.at[c, pl.ds(t * TILE_NODES, TILE_NODES)])


def _sc_layer(uflat, src2, dst2d, z16):
    return pl.kernel(
        _layer_body,
        out_type=jax.ShapeDtypeStruct((NC, NPAD, HALF), jnp.float32),
        mesh=_mesh,
        compiler_params=_sc_params,
        scratch_types=[
            pltpu.VMEM((GRP, 128), jnp.int32),
            pltpu.VMEM((GRP, 128), jnp.int32),
            pltpu.VMEM((GRP, 128, HALF), jnp.float32),
            pltpu.VMEM_SHARED((NPAD, HALF), jnp.float32),
            pltpu.SemaphoreType.DMA,
        ],
    )(uflat, src2, dst2d, z16)


def _pool_body(h_hbm, batch_hbm, zp_hbm, out_hbm, bidx_v, rows_v, pool_sh):
    c = lax.axis_index("c")
    t = lax.axis_index("s")

    @pl.when(t == 0)
    def _():
        pltpu.sync_copy(zp_hbm, pool_sh)

    plsc.subcore_barrier()
    base = t * TILE_NODES

    def chunk(k, _):
        off = base + k * 128
        pltpu.sync_copy(batch_hbm.at[pl.ds(off, 128)], bidx_v.at[0])
        pltpu.sync_copy(h_hbm.at[c, pl.ds(off, 128)], rows_v)
        pltpu.sync_copy(rows_v, pool_sh.at[bidx_v.at[0]], add=True)
        return 0

    lax.fori_loop(0, TILE_NODES // 128, chunk, 0)
    plsc.subcore_barrier()

    @pl.when(t == 0)
    def _():
        pltpu.sync_copy(pool_sh, out_hbm.at[c])


def _sc_pool(h2, batch_pad, zpool):
    return pl.kernel(
        _pool_body,
        out_type=jax.ShapeDtypeStruct((NC, NGPAD, HALF), jnp.float32),
        mesh=_mesh,
        compiler_params=_sc_params,
        scratch_types=[
            pltpu.VMEM((1, 128), jnp.int32),
            pltpu.VMEM((128, HALF), jnp.float32),
            pltpu.VMEM_SHARED((NGPAD, HALF), jnp.float32),
        ],
    )(h2, batch_pad, zpool)


def _prep_kernel(src_ref, out_ref):
    s = src_ref[...]
    out_ref[...] = jnp.stack([s, s + NPAD], axis=0)


def _tc_prep(src2d):
    return pl.pallas_call(
        _prep_kernel,
        grid=(112,),
        in_specs=[pl.BlockSpec((112, 128), lambda i: (i, 0))],
        out_specs=pl.BlockSpec((2, 112, 128), lambda i: (0, i, 0)),
        out_shape=jax.ShapeDtypeStruct((2, EROWS, 128), jnp.int32),
    )(src2d)


def _init_kernel(deg_ref, x_ref, emb_ref, pw0_ref, pb0_ref, pw1_ref, pb1_ref,
                 cw0_ref, dinv_ref, u0_ref):
    deg = deg_ref[0] + deg_ref[1] + 1.0
    dinv = lax.rsqrt(jnp.maximum(deg, 1.0))
    dinv_ref[...] = dinv
    tab = jnp.maximum(emb_ref[...] @ pw0_ref[...] + pb0_ref[...], 0.0)
    tab = jnp.maximum(tab @ pw1_ref[...] + pb1_ref[...], 0.0)
    tab = tab @ cw0_ref[...]
    onehot = (x_ref[...] == lax.broadcasted_iota(jnp.int32, (1, ENC), 1))
    u0 = jnp.dot(onehot.astype(jnp.float32), tab,
                 preferred_element_type=jnp.float32) * dinv
    u0_ref[...] = jnp.stack([u0[:, :HALF], u0[:, HALF:]], axis=0)


def _tc_init(deg, x_pad, emb, pw0, pb0, pw1, pb1, cw0):
    R = NPAD // 32
    return pl.pallas_call(
        _init_kernel,
        grid=(32,),
        in_specs=[
            pl.BlockSpec((2, R, 1), lambda i: (0, i, 0)),
            pl.BlockSpec((R, 1), lambda i: (i, 0)),
            pl.BlockSpec((ENC, CH), lambda i: (0, 0)),
            pl.BlockSpec((CH, CH), lambda i: (0, 0)),
            pl.BlockSpec((1, CH), lambda i: (0, 0)),
            pl.BlockSpec((CH, CH), lambda i: (0, 0)),
            pl.BlockSpec((1, CH), lambda i: (0, 0)),
            pl.BlockSpec((CH, CH), lambda i: (0, 0)),
        ],
        out_specs=[
            pl.BlockSpec((R, 1), lambda i: (i, 0)),
            pl.BlockSpec((2, R, HALF), lambda i: (0, i, 0)),
        ],
        out_shape=[
            jax.ShapeDtypeStruct((NPAD, 1), jnp.float32),
            jax.ShapeDtypeStruct((2, NPAD, HALF), jnp.float32),
        ],
    )(deg, x_pad, emb, pw0, pb0, pw1, pb1, cw0)


def _mid_kernel(s_ref, u_ref, dinv_ref, b_ref, w_ref, out_ref):
    acc = s_ref[...] + u_ref[...]
    h = jnp.concatenate([acc[0], acc[1]], axis=-1)
    h = jnp.maximum(h * dinv_ref[...] + b_ref[...], 0.0)
    un = jnp.dot(h, w_ref[...], preferred_element_type=jnp.float32)
    un = un * dinv_ref[...]
    out_ref[...] = jnp.stack([un[:, :HALF], un[:, HALF:]], axis=0)


def _last_kernel(s_ref, u_ref, dinv_ref, b_ref, out_ref):
    acc = s_ref[...] + u_ref[...]
    h = jnp.concatenate([acc[0], acc[1]], axis=-1)
    h = jnp.maximum(h * dinv_ref[...] + b_ref[...], 0.0)
    out_ref[...] = jnp.stack([h[:, :HALF], h[:, HALF:]], axis=0)


def _tc_mid(s, u, dinv, b, w):
    R = NPAD // 32
    return pl.pallas_call(
        _mid_kernel,
        grid=(32,),
        in_specs=[
            pl.BlockSpec((2, R, HALF), lambda i: (0, i, 0)),
            pl.BlockSpec((2, R, HALF), lambda i: (0, i, 0)),
            pl.BlockSpec((R, 1), lambda i: (i, 0)),
            pl.BlockSpec((1, CH), lambda i: (0, 0)),
            pl.BlockSpec((CH, CH), lambda i: (0, 0)),
        ],
        out_specs=pl.BlockSpec((2, R, HALF), lambda i: (0, i, 0)),
        out_shape=jax.ShapeDtypeStruct((2, NPAD, HALF), jnp.float32),
    )(s, u, dinv, b, w)


def _tc_last(s, u, dinv, b):
    R = NPAD // 32
    return pl.pallas_call(
        _last_kernel,
        grid=(32,),
        in_specs=[
            pl.BlockSpec((2, R, HALF), lambda i: (0, i, 0)),
            pl.BlockSpec((2, R, HALF), lambda i: (0, i, 0)),
            pl.BlockSpec((R, 1), lambda i: (i, 0)),
            pl.BlockSpec((1, CH), lambda i: (0, 0)),
        ],
        out_specs=pl.BlockSpec((2, R, HALF), lambda i: (0, i, 0)),
        out_shape=jax.ShapeDtypeStruct((2, NPAD, HALF), jnp.float32),
    )(s, u, dinv, b)


def _post_kernel(p_ref, w0_ref, b0_ref, w1_ref, b1_ref, pw_ref, pb_ref,
                 out_ref):
    g = jnp.concatenate([p_ref[0, :NG], p_ref[1, :NG]], axis=-1)
    g = jnp.maximum(g @ w0_ref[...] + b0_ref[...], 0.0)
    g = jnp.maximum(g @ w1_ref[...] + b1_ref[...], 0.0)
    props = g @ pw_ref[...] + pb_ref[...]
    out_ref[...] = props[:, 0:1]


def _tc_post(pooled, w0, b0, w1, b1, pw, pb):
    return pl.pallas_call(
        _post_kernel,
        out_shape=jax.ShapeDtypeStruct((NG, 1), jnp.float32),
    )(pooled, w0, b0, w1, b1, pw, pb)


def kernel(x, edge_index, batch, emb, pre_W, pre_b, conv_W, conv_b,
           post_W, post_b, prop_W, prop_b):
    f32 = jnp.float32
    x_pad = jnp.concatenate([x, jnp.zeros((NPAD - N,), jnp.int32)])
    x_pad = x_pad.reshape(NPAD, 1)
    batch_pad = jnp.concatenate(
        [batch, jnp.full((NPAD - N,), NG, jnp.int32)])
    pad_e = EPAD - E
    src_pad = jnp.concatenate(
        [edge_index[0],
         (jnp.arange(pad_e, dtype=jnp.int32) * 997) % N])
    dst_pad = jnp.concatenate(
        [edge_index[1], jnp.full((pad_e,), N, jnp.int32)])
    src2d = src_pad.reshape(EROWS, 128)
    dst2d = dst_pad.reshape(EROWS, 128)
    z1 = jnp.zeros((TILE_NODES,), f32)
    ones = jnp.ones((128,), f32)
    z16 = jnp.zeros((TILE_NODES, HALF), f32)
    zpool = jnp.zeros((NGPAD, HALF), f32)
    b2 = [b.reshape(1, CH) for b in conv_b]
    pb2 = [b.reshape(1, CH) for b in pre_b]

    src2 = _tc_prep(src2d)
    deg = _sc_degree(dst2d, z1, ones)
    dinv, u = _tc_init(deg.reshape(2, NPAD, 1), x_pad, emb,
                       pre_W[0], pb2[0], pre_W[1], pb2[1], conv_W[0])
    h2 = None
    for l in range(6):
        s = _sc_layer(u.reshape(2 * NPAD, HALF), src2, dst2d, z16)
        if l < 5:
            u = _tc_mid(s, u, dinv, b2[l], conv_W[l + 1])
        else:
            h2 = _tc_last(s, u, dinv, b2[l])
    pooled = _sc_pool(h2, batch_pad, zpool)
    out = _tc_post(pooled, post_W[0], post_b[0].reshape(1, -1),
                   post_W[1], post_b[1].reshape(1, -1),
                   prop_W, prop_b.reshape(1, -1))
    return out


# speedup vs baseline: 25.2413x; 1.4354x over previous
import jax
import jax.numpy as jnp
from jax import lax
from jax.experimental import pallas as pl
from jax.experimental.pallas import tpu as pltpu
from jax.experimental.pallas import tpu_sc as plsc

N = 100000
NPAD = 102400
E = 1600000
EPAD = 1605632
EROWS = 12544
CH = 32
HALF = 16
ENC = 128
NG = 256
NGPAD = 264
NC = 2
NS = 16
TILE_NODES = NPAD // NS
LROWS = EROWS // NS
DROWS = EROWS // (NC * NS)
GRP = 8

_mesh = plsc.VectorSubcoreMesh(core_axis_name="c", subcore_axis_name="s",
                               num_cores=NC, num_subcores=NS)
_sc_params = pltpu.CompilerParams(use_tc_tiling_on_sc=False)


def _deg_body(dst_hbm, z1_hbm, ones_hbm, deg_hbm, didx_v, ones_v, deg_sh):
    c = lax.axis_index("c")
    t = lax.axis_index("s")
    pltpu.sync_copy(z1_hbm, deg_sh.at[pl.ds(t * TILE_NODES, TILE_NODES)])
    pltpu.sync_copy(ones_hbm, ones_v)
    plsc.subcore_barrier()
    base = (c * NS + t) * DROWS

    def grp(g, _):
        row0 = base + g * GRP
        pltpu.sync_copy(dst_hbm.at[pl.ds(row0, GRP)], didx_v)
        for j in range(GRP):
            pltpu.sync_copy(ones_v, deg_sh.at[didx_v.at[j]], add=True)
        return 0

    lax.fori_loop(0, DROWS // GRP, grp, 0)
    plsc.subcore_barrier()
    pltpu.sync_copy(deg_sh.at[pl.ds(t * TILE_NODES, TILE_NODES)],
                    deg_hbm.at[c, pl.ds(t * TILE_NODES, TILE_NODES)])


def _sc_degree(dst2d, z1, ones):
    return pl.kernel(
        _deg_body,
        out_type=jax.ShapeDtypeStruct((NC, NPAD), jnp.float32),
        mesh=_mesh,
        compiler_params=_sc_params,
        scratch_types=[
            pltpu.VMEM((GRP, 128), jnp.int32),
            pltpu.VMEM((128,), jnp.float32),
            pltpu.VMEM_SHARED((NPAD,), jnp.float32),
        ],
    )(dst2d, z1, ones)


LGRP = 2
LGROUPS = LROWS // LGRP
BLKG = 4
BROWS = BLKG * LGRP
NBLK = LGROUPS // BLKG
UNROLL = 8


def _layer_body(u_hbm, src2_hbm, dst_hbm, z16_hbm, drows_hbm, s_hbm,
                idx_v, didx_v, rows_v, agg_sh, gsem, ssem, isem):
    c = lax.axis_index("c")
    t = lax.axis_index("s")
    pltpu.sync_copy(z16_hbm, agg_sh.at[pl.ds(t * TILE_NODES, TILE_NODES)])
    base = t * LROWS

    def stage(k, islot):
        r0 = base + k * BROWS
        pltpu.async_copy(src2_hbm.at[c, pl.ds(r0, BROWS)], idx_v.at[islot],
                         isem.at[islot])
        pltpu.async_copy(dst_hbm.at[pl.ds(r0, BROWS)], didx_v.at[islot],
                         isem.at[islot])

    def drain_idx(islot):
        pltpu.make_async_copy(src2_hbm.at[c, pl.ds(base, BROWS)],
                              idx_v.at[islot], isem.at[islot]).wait()
        pltpu.make_async_copy(dst_hbm.at[pl.ds(base, BROWS)],
                              didx_v.at[islot], isem.at[islot]).wait()

    def drain_rows(b, sem):
        pltpu.make_async_copy(drows_hbm, rows_v.at[b], sem.at[b]).wait()

    def fire_gathers(b, g, islot, brow):
        for j in range(LGRP):
            pltpu.async_copy(u_hbm.at[idx_v.at[islot, brow + j]],
                             rows_v.at[b, j], gsem.at[b])
        del g

    def fire_scatters(b, islot, brow):
        for j in range(LGRP):
            pltpu.async_copy(rows_v.at[b, j],
                             agg_sh.at[didx_v.at[islot, brow + j]],
                             ssem.at[b], add=True)

    stage(0, 0)
    plsc.subcore_barrier()

    def outer(i, _):
        for v in range(UNROLL):
            g = i * UNROLL + v
            b = v % 4
            islot_a = (v // BLKG) % 2
            brow_a = (v % BLKG) * LGRP
            if v % BLKG == 0:
                drain_idx(islot_a)

            @pl.when(g >= 4)
            def _():
                drain_rows(b, ssem)

            fire_gathers(b, g, islot_a, brow_a)
            if v % BLKG == 3:
                blk = g // BLKG + 1

                @pl.when(blk < NBLK)
                def _():
                    stage(blk, (islot_a + 1) % 2)

            bp = (v - 2) % 4
            vb = (v - 2) % UNROLL
            islot_b = (vb // BLKG) % 2
            brow_b = (vb % BLKG) * LGRP

            @pl.when(g >= 2)
            def _():
                drain_rows(bp, gsem)
                fire_scatters(bp, islot_b, brow_b)

        return 0

    lax.fori_loop(0, LGROUPS // UNROLL, outer, 0)
    for gp in (LGROUPS - 2, LGROUPS - 1):
        v = gp % UNROLL
        drain_rows(v % 4, gsem)
        fire_scatters(v % 4, (v // BLKG) % 2, (v % BLKG) * LGRP)
    for b in range(4):
        drain_rows(b, ssem)
    plsc.subcore_barrier()
    pltpu.sync_copy(agg_sh.at[pl.ds(t * TILE_NODES, TILE_NODES)],
                    s_hbm.at[c, pl.ds(t * TILE_NODES, TILE_NODES)])


def _sc_layer(uflat, src2, dst2d, z16, drows):
    return pl.kernel(
        _layer_body,
        out_type=jax.ShapeDtypeStruct((NC, NPAD, HALF), jnp.float32),
        mesh=_mesh,
        compiler_params=_sc_params,
        scratch_types=[
            pltpu.VMEM((2, BROWS, 128), jnp.int32),
            pltpu.VMEM((2, BROWS, 128), jnp.int32),
            pltpu.VMEM((4, LGRP, 128, HALF), jnp.float32),
            pltpu.VMEM_SHARED((NPAD, HALF), jnp.float32),
            pltpu.SemaphoreType.DMA((4,)),
            pltpu.SemaphoreType.DMA((4,)),
            pltpu.SemaphoreType.DMA((2,)),
        ],
    )(uflat, src2, dst2d, z16, drows)


def _pool_body(h_hbm, batch_hbm, zp_hbm, out_hbm, bidx_v, rows_v, pool_sh):
    c = lax.axis_index("c")
    t = lax.axis_index("s")

    @pl.when(t == 0)
    def _():
        pltpu.sync_copy(zp_hbm, pool_sh)

    plsc.subcore_barrier()
    base = t * TILE_NODES

    def chunk(k, _):
        off = base + k * 128
        pltpu.sync_copy(batch_hbm.at[pl.ds(off, 128)], bidx_v.at[0])
        pltpu.sync_copy(h_hbm.at[c, pl.ds(off, 128)], rows_v)
        pltpu.sync_copy(rows_v, pool_sh.at[bidx_v.at[0]], add=True)
        return 0

    lax.fori_loop(0, TILE_NODES // 128, chunk, 0)
    plsc.subcore_barrier()

    @pl.when(t == 0)
    def _():
        pltpu.sync_copy(pool_sh, out_hbm.at[c])


def _sc_pool(h2, batch_pad, zpool):
    return pl.kernel(
        _pool_body,
        out_type=jax.ShapeDtypeStruct((NC, NGPAD, HALF), jnp.float32),
        mesh=_mesh,
        compiler_params=_sc_params,
        scratch_types=[
            pltpu.VMEM((1, 128), jnp.int32),
            pltpu.VMEM((128, HALF), jnp.float32),
            pltpu.VMEM_SHARED((NGPAD, HALF), jnp.float32),
        ],
    )(h2, batch_pad, zpool)


def _prep_kernel(src_ref, out_ref):
    s = src_ref[...]
    out_ref[...] = jnp.stack([s, s + NPAD], axis=0)


def _tc_prep(src2d):
    return pl.pallas_call(
        _prep_kernel,
        grid=(112,),
        in_specs=[pl.BlockSpec((112, 128), lambda i: (i, 0))],
        out_specs=pl.BlockSpec((2, 112, 128), lambda i: (0, i, 0)),
        out_shape=jax.ShapeDtypeStruct((2, EROWS, 128), jnp.int32),
    )(src2d)


def _init_kernel(deg_ref, x_ref, emb_ref, pw0_ref, pb0_ref, pw1_ref, pb1_ref,
                 cw0_ref, dinv_ref, u0_ref):
    deg = deg_ref[0] + deg_ref[1] + 1.0
    dinv = lax.rsqrt(jnp.maximum(deg, 1.0))
    dinv_ref[...] = dinv
    tab = jnp.maximum(emb_ref[...] @ pw0_ref[...] + pb0_ref[...], 0.0)
    tab = jnp.maximum(tab @ pw1_ref[...] + pb1_ref[...], 0.0)
    tab = tab @ cw0_ref[...]
    onehot = (x_ref[...] == lax.broadcasted_iota(jnp.int32, (1, ENC), 1))
    u0 = jnp.dot(onehot.astype(jnp.float32), tab,
                 preferred_element_type=jnp.float32) * dinv
    u0_ref[...] = jnp.stack([u0[:, :HALF], u0[:, HALF:]], axis=0)


def _tc_init(deg, x_pad, emb, pw0, pb0, pw1, pb1, cw0):
    R = NPAD // 32
    return pl.pallas_call(
        _init_kernel,
        grid=(32,),
        in_specs=[
            pl.BlockSpec((2, R, 1), lambda i: (0, i, 0)),
            pl.BlockSpec((R, 1), lambda i: (i, 0)),
            pl.BlockSpec((ENC, CH), lambda i: (0, 0)),
            pl.BlockSpec((CH, CH), lambda i: (0, 0)),
            pl.BlockSpec((1, CH), lambda i: (0, 0)),
            pl.BlockSpec((CH, CH), lambda i: (0, 0)),
            pl.BlockSpec((1, CH), lambda i: (0, 0)),
            pl.BlockSpec((CH, CH), lambda i: (0, 0)),
        ],
        out_specs=[
            pl.BlockSpec((R, 1), lambda i: (i, 0)),
            pl.BlockSpec((2, R, HALF), lambda i: (0, i, 0)),
        ],
        out_shape=[
            jax.ShapeDtypeStruct((NPAD, 1), jnp.float32),
            jax.ShapeDtypeStruct((2, NPAD, HALF), jnp.float32),
        ],
    )(deg, x_pad, emb, pw0, pb0, pw1, pb1, cw0)


def _mid_kernel(s_ref, u_ref, dinv_ref, b_ref, w_ref, out_ref):
    acc = s_ref[...] + u_ref[...]
    h = jnp.concatenate([acc[0], acc[1]], axis=-1)
    h = jnp.maximum(h * dinv_ref[...] + b_ref[...], 0.0)
    un = jnp.dot(h, w_ref[...], preferred_element_type=jnp.float32)
    un = un * dinv_ref[...]
    out_ref[...] = jnp.stack([un[:, :HALF], un[:, HALF:]], axis=0)


def _last_kernel(s_ref, u_ref, dinv_ref, b_ref, out_ref):
    acc = s_ref[...] + u_ref[...]
    h = jnp.concatenate([acc[0], acc[1]], axis=-1)
    h = jnp.maximum(h * dinv_ref[...] + b_ref[...], 0.0)
    out_ref[...] = jnp.stack([h[:, :HALF], h[:, HALF:]], axis=0)


def _tc_mid(s, u, dinv, b, w):
    R = NPAD // 32
    return pl.pallas_call(
        _mid_kernel,
        grid=(32,),
        in_specs=[
            pl.BlockSpec((2, R, HALF), lambda i: (0, i, 0)),
            pl.BlockSpec((2, R, HALF), lambda i: (0, i, 0)),
            pl.BlockSpec((R, 1), lambda i: (i, 0)),
            pl.BlockSpec((1, CH), lambda i: (0, 0)),
            pl.BlockSpec((CH, CH), lambda i: (0, 0)),
        ],
        out_specs=pl.BlockSpec((2, R, HALF), lambda i: (0, i, 0)),
        out_shape=jax.ShapeDtypeStruct((2, NPAD, HALF), jnp.float32),
    )(s, u, dinv, b, w)


def _tc_last(s, u, dinv, b):
    R = NPAD // 32
    return pl.pallas_call(
        _last_kernel,
        grid=(32,),
        in_specs=[
            pl.BlockSpec((2, R, HALF), lambda i: (0, i, 0)),
            pl.BlockSpec((2, R, HALF), lambda i: (0, i, 0)),
            pl.BlockSpec((R, 1), lambda i: (i, 0)),
            pl.BlockSpec((1, CH), lambda i: (0, 0)),
        ],
        out_specs=pl.BlockSpec((2, R, HALF), lambda i: (0, i, 0)),
        out_shape=jax.ShapeDtypeStruct((2, NPAD, HALF), jnp.float32),
    )(s, u, dinv, b)


def _post_kernel(p_ref, w0_ref, b0_ref, w1_ref, b1_ref, pw_ref, pb_ref,
                 out_ref):
    g = jnp.concatenate([p_ref[0, :NG], p_ref[1, :NG]], axis=-1)
    g = jnp.maximum(g @ w0_ref[...] + b0_ref[...], 0.0)
    g = jnp.maximum(g @ w1_ref[...] + b1_ref[...], 0.0)
    props = g @ pw_ref[...] + pb_ref[...]
    out_ref[...] = props[:, 0:1]


def _tc_post(pooled, w0, b0, w1, b1, pw, pb):
    return pl.pallas_call(
        _post_kernel,
        out_shape=jax.ShapeDtypeStruct((NG, 1), jnp.float32),
    )(pooled, w0, b0, w1, b1, pw, pb)


def kernel(x, edge_index, batch, emb, pre_W, pre_b, conv_W, conv_b,
           post_W, post_b, prop_W, prop_b):
    f32 = jnp.float32
    x_pad = jnp.concatenate([x, jnp.zeros((NPAD - N,), jnp.int32)])
    x_pad = x_pad.reshape(NPAD, 1)
    batch_pad = jnp.concatenate(
        [batch, jnp.full((NPAD - N,), NG, jnp.int32)])
    pad_e = EPAD - E
    src_pad = jnp.concatenate(
        [edge_index[0],
         (jnp.arange(pad_e, dtype=jnp.int32) * 997) % N])
    dst_pad = jnp.concatenate(
        [edge_index[1], jnp.full((pad_e,), N, jnp.int32)])
    src2d = src_pad.reshape(EROWS, 128)
    dst2d = dst_pad.reshape(EROWS, 128)
    z1 = jnp.zeros((TILE_NODES,), f32)
    ones = jnp.ones((128,), f32)
    z16 = jnp.zeros((TILE_NODES, HALF), f32)
    drows = jnp.zeros((LGRP, 128, HALF), f32)
    zpool = jnp.zeros((NGPAD, HALF), f32)
    b2 = [b.reshape(1, CH) for b in conv_b]
    pb2 = [b.reshape(1, CH) for b in pre_b]

    src2 = _tc_prep(src2d)
    deg = _sc_degree(dst2d, z1, ones)
    dinv, u = _tc_init(deg.reshape(2, NPAD, 1), x_pad, emb,
                       pre_W[0], pb2[0], pre_W[1], pb2[1], conv_W[0])
    h2 = None
    for l in range(6):
        s = _sc_layer(u.reshape(2 * NPAD, HALF), src2, dst2d, z16, drows)
        if l < 5:
            u = _tc_mid(s, u, dinv, b2[l], conv_W[l + 1])
        else:
            h2 = _tc_last(s, u, dinv, b2[l])
    pooled = _sc_pool(h2, batch_pad, zpool)
    out = _tc_post(pooled, post_W[0], post_b[0].reshape(1, -1),
                   post_W[1], post_b[1].reshape(1, -1),
                   prop_W, prop_b.reshape(1, -1))
    return out


# speedup vs baseline: 47.0080x; 1.8623x over previous
import jax
import jax.numpy as jnp
from jax import lax
from jax.experimental import pallas as pl
from jax.experimental.pallas import tpu as pltpu
from jax.experimental.pallas import tpu_sc as plsc

N = 100000
NPAD = 102400
E = 1600000
EPAD = 1605632
EROWS = 12544
CH = 32
HALF = 16
ENC = 128
NG = 256
NGPAD = 264
NC = 2
NS = 16
NR = NPAD // 8
TILE_NODES = NPAD // NS
LROWS = EROWS // NS
DROWS = EROWS // (NC * NS)
GRP = 8

_mesh = plsc.VectorSubcoreMesh(core_axis_name="c", subcore_axis_name="s",
                               num_cores=NC, num_subcores=NS)
_sc_params = pltpu.CompilerParams(use_tc_tiling_on_sc=False)


def _deg_body(dst_hbm, xoff_hbm, tab_hbm, z1_hbm, ones_hbm, deg_hbm,
              t1x_hbm, didx_v, ones_v, xi_v, rows2_v, deg_sh, dsem):
    c = lax.axis_index("c")
    t = lax.axis_index("s")
    pltpu.sync_copy(z1_hbm, deg_sh.at[pl.ds(t * TILE_NODES, TILE_NODES)])
    pltpu.sync_copy(ones_hbm, ones_v)
    XCH = TILE_NODES // 128
    pltpu.sync_copy(xoff_hbm.at[c, pl.ds(t * XCH, XCH)], xi_v)

    def fire(ch, slot):
        pltpu.async_copy(tab_hbm.at[xi_v.at[ch]], rows2_v.at[slot],
                         dsem.at[slot])

    fire(0, 0)

    def chunk2(i, _):
        for b in range(2):
            ch = i * 2 + b

            @pl.when(ch + 1 < XCH)
            def _():
                fire(ch + 1, (b + 1) % 2)

            pltpu.make_async_copy(tab_hbm.at[xi_v.at[0]], rows2_v.at[b],
                                  dsem.at[b]).wait()
            off = t * TILE_NODES + ch * 128
            pltpu.sync_copy(rows2_v.at[b], t1x_hbm.at[c, pl.ds(off, 128)])
        return 0

    lax.fori_loop(0, XCH // 2, chunk2, 0)

    base = (c * NS + t) * DROWS

    def grp(g, _):
        row0 = base + g * GRP
        pltpu.sync_copy(dst_hbm.at[pl.ds(row0, GRP)], didx_v)
        for j in range(GRP):
            pltpu.sync_copy(ones_v, deg_sh.at[didx_v.at[j]], add=True)
        return 0

    lax.fori_loop(0, DROWS // GRP, grp, 0)
    plsc.subcore_barrier()
    pltpu.sync_copy(deg_sh.at[pl.ds(t * TILE_NODES, TILE_NODES)],
                    deg_hbm.at[c, pl.ds(t * TILE_NODES, TILE_NODES)])


def _sc_degree(dst2d, xoff, tab, z1, ones):
    return pl.kernel(
        _deg_body,
        out_type=[
            jax.ShapeDtypeStruct((NC, NPAD), jnp.float32),
            jax.ShapeDtypeStruct((NC, NPAD, HALF), jnp.float32),
        ],
        mesh=_mesh,
        compiler_params=_sc_params,
        scratch_types=[
            pltpu.VMEM((GRP, 128), jnp.int32),
            pltpu.VMEM((128,), jnp.float32),
            pltpu.VMEM((TILE_NODES // 128, 128), jnp.int32),
            pltpu.VMEM((2, 128, HALF), jnp.float32),
            pltpu.VMEM_SHARED((NPAD,), jnp.float32),
            pltpu.SemaphoreType.DMA((2,)),
        ],
    )(dst2d, xoff, tab, z1, ones)


LGRP = 2
LGROUPS = LROWS // LGRP
BLKG = 4
BROWS = BLKG * LGRP
NBLK = LGROUPS // BLKG
UNROLL = 8


def _layer_body(u_hbm, src2_hbm, dst_hbm, z16_hbm, drows_hbm, s_hbm,
                idx_v, didx_v, rows_v, agg_sh, gsem, ssem, isem):
    c = lax.axis_index("c")
    t = lax.axis_index("s")
    pltpu.sync_copy(z16_hbm, agg_sh.at[pl.ds(t * TILE_NODES, TILE_NODES)])
    base = t * LROWS

    def stage(k, islot):
        r0 = base + k * BROWS
        pltpu.async_copy(src2_hbm.at[c, pl.ds(r0, BROWS)], idx_v.at[islot],
                         isem.at[islot])
        pltpu.async_copy(dst_hbm.at[pl.ds(r0, BROWS)], didx_v.at[islot],
                         isem.at[islot])

    def drain_idx(islot):
        pltpu.make_async_copy(src2_hbm.at[c, pl.ds(base, BROWS)],
                              idx_v.at[islot], isem.at[islot]).wait()
        pltpu.make_async_copy(dst_hbm.at[pl.ds(base, BROWS)],
                              didx_v.at[islot], isem.at[islot]).wait()

    def drain_rows(b, sem):
        pltpu.make_async_copy(drows_hbm, rows_v.at[b], sem.at[b]).wait()

    def fire_gathers(b, islot, brow):
        for j in range(LGRP):
            pltpu.async_copy(u_hbm.at[idx_v.at[islot, brow + j]],
                             rows_v.at[b, j], gsem.at[b])

    def fire_scatters(b, islot, brow):
        for j in range(LGRP):
            pltpu.async_copy(rows_v.at[b, j],
                             agg_sh.at[didx_v.at[islot, brow + j]],
                             ssem.at[b], add=True)

    stage(0, 0)
    plsc.subcore_barrier()

    def outer(i, _):
        for v in range(UNROLL):
            g = i * UNROLL + v
            b = v % 4
            islot_a = (v // BLKG) % 2
            brow_a = (v % BLKG) * LGRP
            if v % BLKG == 0:
                drain_idx(islot_a)

            @pl.when(g >= 4)
            def _():
                drain_rows(b, ssem)

            fire_gathers(b, islot_a, brow_a)
            if v % BLKG == 3:
                blk = g // BLKG + 1

                @pl.when(blk < NBLK)
                def _():
                    stage(blk, (islot_a + 1) % 2)

            bp = (v - 2) % 4
            vb = (v - 2) % UNROLL
            islot_b = (vb // BLKG) % 2
            brow_b = (vb % BLKG) * LGRP

            @pl.when(g >= 2)
            def _():
                drain_rows(bp, gsem)
                fire_scatters(bp, islot_b, brow_b)

        return 0

    lax.fori_loop(0, LGROUPS // UNROLL, outer, 0)
    for gp in (LGROUPS - 2, LGROUPS - 1):
        v = gp % UNROLL
        drain_rows(v % 4, gsem)
        fire_scatters(v % 4, (v // BLKG) % 2, (v % BLKG) * LGRP)
    for b in range(4):
        drain_rows(b, ssem)
    plsc.subcore_barrier()
    pltpu.sync_copy(agg_sh.at[pl.ds(t * TILE_NODES, TILE_NODES)],
                    s_hbm.at[c, pl.ds(t * TILE_NODES, TILE_NODES)])


def _sc_layer(uflat, src2, dst2d, z16, drows):
    return pl.kernel(
        _layer_body,
        out_type=jax.ShapeDtypeStruct((NC, NPAD, HALF), jnp.float32),
        mesh=_mesh,
        compiler_params=_sc_params,
        scratch_types=[
            pltpu.VMEM((2, BROWS, 128), jnp.int32),
            pltpu.VMEM((2, BROWS, 128), jnp.int32),
            pltpu.VMEM((4, LGRP, 128, HALF), jnp.float32),
            pltpu.VMEM_SHARED((NPAD, HALF), jnp.float32),
            pltpu.SemaphoreType.DMA((4,)),
            pltpu.SemaphoreType.DMA((4,)),
            pltpu.SemaphoreType.DMA((2,)),
        ],
    )(uflat, src2, dst2d, z16, drows)


def _pool_body(h_hbm, batch_hbm, zp_hbm, out_hbm, bidx_v, rows_v, pool_sh):
    c = lax.axis_index("c")
    t = lax.axis_index("s")

    @pl.when(t == 0)
    def _():
        pltpu.sync_copy(zp_hbm, pool_sh)

    plsc.subcore_barrier()
    base = t * TILE_NODES

    def chunk(k, _):
        off = base + k * 128
        pltpu.sync_copy(batch_hbm.at[pl.ds(off, 128)], bidx_v.at[0])
        pltpu.sync_copy(h_hbm.at[pl.ds(c * NPAD + off, 128)], rows_v)
        pltpu.sync_copy(rows_v, pool_sh.at[bidx_v.at[0]], add=True)
        return 0

    lax.fori_loop(0, TILE_NODES // 128, chunk, 0)
    plsc.subcore_barrier()

    @pl.when(t == 0)
    def _():
        pltpu.sync_copy(pool_sh, out_hbm.at[c])


def _sc_pool(hflat, batch_pad, zpool):
    return pl.kernel(
        _pool_body,
        out_type=jax.ShapeDtypeStruct((NC, NGPAD, HALF), jnp.float32),
        mesh=_mesh,
        compiler_params=_sc_params,
        scratch_types=[
            pltpu.VMEM((1, 128), jnp.int32),
            pltpu.VMEM((128, HALF), jnp.float32),
            pltpu.VMEM_SHARED((NGPAD, HALF), jnp.float32),
        ],
    )(hflat, batch_pad, zpool)


def _table_kernel(emb_ref, pw0_ref, pb0_ref, pw1_ref, pb1_ref, cw0_ref,
                  out_ref):
    tab = jnp.maximum(emb_ref[...] @ pw0_ref[...] + pb0_ref[...], 0.0)
    tab = jnp.maximum(tab @ pw1_ref[...] + pb1_ref[...], 0.0)
    tab = tab @ cw0_ref[...]
    out_ref[...] = jnp.concatenate([tab[:, :HALF], tab[:, HALF:]], axis=0)


def _tc_table(emb, pw0, pb0, pw1, pb1, cw0):
    return pl.pallas_call(
        _table_kernel,
        out_shape=jax.ShapeDtypeStruct((2 * ENC, HALF), jnp.float32),
    )(emb, pw0, pb0, pw1, pb1, cw0)


def _xprep_kernel(x_ref, out_ref):
    xx = x_ref[...]
    out_ref[...] = jnp.stack([xx, xx + ENC], axis=0)


def _tc_xprep(x2d):
    return pl.pallas_call(
        _xprep_kernel,
        out_shape=jax.ShapeDtypeStruct((2, NPAD // 128, 128), jnp.int32),
    )(x2d)


def _prep_kernel(src_ref, out_ref):
    s = src_ref[...]
    out_ref[...] = jnp.stack([s, s + NPAD], axis=0)


def _tc_prep(src2d):
    return pl.pallas_call(
        _prep_kernel,
        grid=(112,),
        in_specs=[pl.BlockSpec((112, 128), lambda i: (i, 0))],
        out_specs=pl.BlockSpec((2, 112, 128), lambda i: (0, i, 0)),
        out_shape=jax.ShapeDtypeStruct((2, EROWS, 128), jnp.int32),
    )(src2d)


def _init_kernel(deg_ref, t1x_ref, dinv_ref, u0_ref):
    deg8 = deg_ref[0] + deg_ref[1] + 1.0
    dinv8 = lax.rsqrt(jnp.maximum(deg8, 1.0))
    ii = lax.broadcasted_iota(jnp.int32, (8, 128), 0)
    jj = lax.broadcasted_iota(jnp.int32, (8, 128), 1)
    rep = (jj // HALF == ii).astype(jnp.float32)
    dinv = jnp.dot(dinv8, rep, preferred_element_type=jnp.float32)
    dinv_ref[...] = dinv
    u0_ref[...] = t1x_ref[...] * dinv[None]


def _tc_init(deg8, t1x):
    R = NR // 32
    return pl.pallas_call(
        _init_kernel,
        grid=(32,),
        in_specs=[
            pl.BlockSpec((2, R, 8), lambda i: (0, i, 0)),
            pl.BlockSpec((2, R, 128), lambda i: (0, i, 0)),
        ],
        out_specs=[
            pl.BlockSpec((R, 128), lambda i: (i, 0)),
            pl.BlockSpec((2, R, 128), lambda i: (0, i, 0)),
        ],
        out_shape=[
            jax.ShapeDtypeStruct((NR, 128), jnp.float32),
            jax.ShapeDtypeStruct((2, NR, 128), jnp.float32),
        ],
    )(deg8, t1x)


def _mid_kernel(s_ref, u_ref, dinv_ref, bl_ref, bh_ref,
                kaa_ref, kba_ref, kab_ref, kbb_ref, out_ref):
    dinv = dinv_ref[...]
    hl = jnp.maximum(dinv * (s_ref[0] + u_ref[0]) + bl_ref[...], 0.0)
    hh = jnp.maximum(dinv * (s_ref[1] + u_ref[1]) + bh_ref[...], 0.0)
    ul = jnp.dot(hl, kaa_ref[...], preferred_element_type=jnp.float32)
    ul += jnp.dot(hh, kba_ref[...], preferred_element_type=jnp.float32)
    uh = jnp.dot(hl, kab_ref[...], preferred_element_type=jnp.float32)
    uh += jnp.dot(hh, kbb_ref[...], preferred_element_type=jnp.float32)
    out_ref[...] = jnp.stack([ul * dinv, uh * dinv], axis=0)


def _last_kernel(s_ref, u_ref, dinv_ref, bl_ref, bh_ref, out_ref):
    dinv = dinv_ref[...]
    hl = jnp.maximum(dinv * (s_ref[0] + u_ref[0]) + bl_ref[...], 0.0)
    hh = jnp.maximum(dinv * (s_ref[1] + u_ref[1]) + bh_ref[...], 0.0)
    out_ref[...] = jnp.stack([hl, hh], axis=0)


def _tc_mid(s, u, dinv, bl, bh, ks):
    R = NR // 32
    return pl.pallas_call(
        _mid_kernel,
        grid=(32,),
        in_specs=[
            pl.BlockSpec((2, R, 128), lambda i: (0, i, 0)),
            pl.BlockSpec((2, R, 128), lambda i: (0, i, 0)),
            pl.BlockSpec((R, 128), lambda i: (i, 0)),
            pl.BlockSpec((1, 128), lambda i: (0, 0)),
            pl.BlockSpec((1, 128), lambda i: (0, 0)),
            pl.BlockSpec((128, 128), lambda i: (0, 0)),
            pl.BlockSpec((128, 128), lambda i: (0, 0)),
            pl.BlockSpec((128, 128), lambda i: (0, 0)),
            pl.BlockSpec((128, 128), lambda i: (0, 0)),
        ],
        out_specs=pl.BlockSpec((2, R, 128), lambda i: (0, i, 0)),
        out_shape=jax.ShapeDtypeStruct((2, NR, 128), jnp.float32),
    )(s, u, dinv, bl, bh, *ks)


def _tc_last(s, u, dinv, bl, bh):
    R = NR // 32
    return pl.pallas_call(
        _last_kernel,
        grid=(32,),
        in_specs=[
            pl.BlockSpec((2, R, 128), lambda i: (0, i, 0)),
            pl.BlockSpec((2, R, 128), lambda i: (0, i, 0)),
            pl.BlockSpec((R, 128), lambda i: (i, 0)),
            pl.BlockSpec((1, 128), lambda i: (0, 0)),
            pl.BlockSpec((1, 128), lambda i: (0, 0)),
        ],
        out_specs=pl.BlockSpec((2, R, 128), lambda i: (0, i, 0)),
        out_shape=jax.ShapeDtypeStruct((2, NR, 128), jnp.float32),
    )(s, u, dinv, bl, bh)


def _post_kernel(p_ref, w0_ref, b0_ref, w1_ref, b1_ref, pw_ref, pb_ref,
                 out_ref):
    g = jnp.concatenate([p_ref[0, :NG], p_ref[1, :NG]], axis=-1)
    g = jnp.maximum(g @ w0_ref[...] + b0_ref[...], 0.0)
    g = jnp.maximum(g @ w1_ref[...] + b1_ref[...], 0.0)
    props = g @ pw_ref[...] + pb_ref[...]
    out_ref[...] = props[:, 0:1]


def _tc_post(pooled, w0, b0, w1, b1, pw, pb):
    return pl.pallas_call(
        _post_kernel,
        out_shape=jax.ShapeDtypeStruct((NG, 1), jnp.float32),
    )(pooled, w0, b0, w1, b1, pw, pb)


def kernel(x, edge_index, batch, emb, pre_W, pre_b, conv_W, conv_b,
           post_W, post_b, prop_W, prop_b):
    f32 = jnp.float32
    x_pad = jnp.concatenate([x, jnp.zeros((NPAD - N,), jnp.int32)])
    batch_pad = jnp.concatenate(
        [batch, jnp.full((NPAD - N,), NG, jnp.int32)])
    pad_e = EPAD - E
    src_pad = jnp.concatenate(
        [edge_index[0],
         (jnp.arange(pad_e, dtype=jnp.int32) * 997) % N])
    dst_pad = jnp.concatenate(
        [edge_index[1], jnp.full((pad_e,), N, jnp.int32)])
    src2d = src_pad.reshape(EROWS, 128)
    dst2d = dst_pad.reshape(EROWS, 128)
    z1 = jnp.zeros((TILE_NODES,), f32)
    ones = jnp.ones((128,), f32)
    z16 = jnp.zeros((TILE_NODES, HALF), f32)
    drows = jnp.zeros((LGRP, 128, HALF), f32)
    zpool = jnp.zeros((NGPAD, HALF), f32)
    eye8 = jnp.eye(8, dtype=f32)
    ks = [[jnp.kron(eye8, w[:HALF, :HALF]), jnp.kron(eye8, w[HALF:, :HALF]),
           jnp.kron(eye8, w[:HALF, HALF:]), jnp.kron(eye8, w[HALF:, HALF:])]
          for w in conv_W[1:]]
    bl = [jnp.tile(b[:HALF], 8).reshape(1, 128) for b in conv_b]
    bh = [jnp.tile(b[HALF:], 8).reshape(1, 128) for b in conv_b]
    pb2 = [b.reshape(1, CH) for b in pre_b]

    tab = _tc_table(emb, pre_W[0], pb2[0], pre_W[1], pb2[1], conv_W[0])
    xoff = _tc_xprep(x_pad.reshape(NPAD // 128, 128))
    xoff = xoff.reshape(2, NPAD // 128, 128)
    src2 = _tc_prep(src2d)
    deg, t1x = _sc_degree(dst2d, xoff, tab, z1, ones)
    dinv, u = _tc_init(deg.reshape(2, NR, 8), t1x.reshape(2, NR, 128))
    h2 = None
    for l in range(6):
        s = _sc_layer(u.reshape(2 * NPAD, HALF), src2, dst2d, z16, drows)
        spk = s.reshape(2, NR, 128)
        if l < 5:
            u = _tc_mid(spk, u, dinv, bl[l], bh[l], ks[l])
        else:
            h2 = _tc_last(spk, u, dinv, bl[l], bh[l])
    pooled = _sc_pool(h2.reshape(2 * NPAD, HALF), batch_pad, zpool)
    out = _tc_post(pooled, post_W[0], post_b[0].reshape(1, -1),
                   post_W[1], post_b[1].reshape(1, -1),
                   prop_W, prop_b.reshape(1, -1))
    return out


# speedup vs baseline: 47.6827x; 1.0144x over previous
import jax
import jax.numpy as jnp
from jax import lax
from jax.experimental import pallas as pl
from jax.experimental.pallas import tpu as pltpu
from jax.experimental.pallas import tpu_sc as plsc

N = 100000
NPAD = 102400
E = 1600000
EPAD = 1605632
EROWS = 12544
CH = 32
HALF = 16
ENC = 128
NG = 256
NGPAD = 264
NC = 2
NS = 16
NR = NPAD // 8
TILE_NODES = NPAD // NS
LROWS = EROWS // NS
DROWS = EROWS // (NC * NS)
GRP = 8

_mesh = plsc.VectorSubcoreMesh(core_axis_name="c", subcore_axis_name="s",
                               num_cores=NC, num_subcores=NS)
_sc_params = pltpu.CompilerParams(use_tc_tiling_on_sc=False)


def _deg_body(dst_hbm, xoff_hbm, tab_hbm, z16_hbm, ones_hbm, deg_hbm,
              t1x_hbm, didx_v, ones_v, xi_v, rows2_v, deg_sh, dsem):
    c = lax.axis_index("c")
    t = lax.axis_index("s")
    pltpu.sync_copy(z16_hbm, deg_sh.at[pl.ds(t * TILE_NODES, TILE_NODES)])
    pltpu.sync_copy(ones_hbm, ones_v)
    XCH = TILE_NODES // 128
    pltpu.sync_copy(xoff_hbm.at[c, pl.ds(t * XCH, XCH)], xi_v)

    def fire(ch, slot):
        pltpu.async_copy(tab_hbm.at[xi_v.at[ch]], rows2_v.at[slot],
                         dsem.at[slot])

    fire(0, 0)

    def chunk2(i, _):
        for b in range(2):
            ch = i * 2 + b

            @pl.when(ch + 1 < XCH)
            def _():
                fire(ch + 1, (b + 1) % 2)

            pltpu.make_async_copy(tab_hbm.at[xi_v.at[0]], rows2_v.at[b],
                                  dsem.at[b]).wait()
            off = t * TILE_NODES + ch * 128
            pltpu.sync_copy(rows2_v.at[b], t1x_hbm.at[c, pl.ds(off, 128)])
        return 0

    lax.fori_loop(0, XCH // 2, chunk2, 0)

    base = (c * NS + t) * DROWS

    def grp(g, _):
        row0 = base + g * GRP
        pltpu.sync_copy(dst_hbm.at[pl.ds(row0, GRP)], didx_v)
        for j in range(GRP):
            pltpu.sync_copy(ones_v, deg_sh.at[didx_v.at[j]], add=True)
        return 0

    lax.fori_loop(0, DROWS // GRP, grp, 0)
    plsc.subcore_barrier()
    pltpu.sync_copy(deg_sh.at[pl.ds(t * TILE_NODES, TILE_NODES)],
                    deg_hbm.at[c, pl.ds(t * TILE_NODES, TILE_NODES)])


def _sc_degree(dst2d, xoff, tab, z16, ones16):
    return pl.kernel(
        _deg_body,
        out_type=[
            jax.ShapeDtypeStruct((NC, NPAD, HALF), jnp.float32),
            jax.ShapeDtypeStruct((NC, NPAD, HALF), jnp.float32),
        ],
        mesh=_mesh,
        compiler_params=_sc_params,
        scratch_types=[
            pltpu.VMEM((GRP, 128), jnp.int32),
            pltpu.VMEM((128, HALF), jnp.float32),
            pltpu.VMEM((TILE_NODES // 128, 128), jnp.int32),
            pltpu.VMEM((2, 128, HALF), jnp.float32),
            pltpu.VMEM_SHARED((NPAD, HALF), jnp.float32),
            pltpu.SemaphoreType.DMA((2,)),
        ],
    )(dst2d, xoff, tab, z16, ones16)


LGRP = 2
LGROUPS = LROWS // LGRP
BLKG = 4
BROWS = BLKG * LGRP
NBLK = LGROUPS // BLKG
UNROLL = 8


def _layer_body(u_hbm, src2_hbm, dst_hbm, z16_hbm, drows_hbm, s_hbm,
                idx_v, didx_v, rows_v, agg_sh, gsem, ssem, isem):
    c = lax.axis_index("c")
    t = lax.axis_index("s")
    pltpu.sync_copy(z16_hbm, agg_sh.at[pl.ds(t * TILE_NODES, TILE_NODES)])
    base = t * LROWS

    def stage(k, islot):
        r0 = base + k * BROWS
        pltpu.async_copy(src2_hbm.at[c, pl.ds(r0, BROWS)], idx_v.at[islot],
                         isem.at[islot])
        pltpu.async_copy(dst_hbm.at[pl.ds(r0, BROWS)], didx_v.at[islot],
                         isem.at[islot])

    def drain_idx(islot):
        pltpu.make_async_copy(src2_hbm.at[c, pl.ds(base, BROWS)],
                              idx_v.at[islot], isem.at[islot]).wait()
        pltpu.make_async_copy(dst_hbm.at[pl.ds(base, BROWS)],
                              didx_v.at[islot], isem.at[islot]).wait()

    def drain_rows(b, sem):
        pltpu.make_async_copy(drows_hbm, rows_v.at[b], sem.at[b]).wait()

    def fire_gathers(b, islot, brow):
        for j in range(LGRP):
            pltpu.async_copy(u_hbm.at[idx_v.at[islot, brow + j]],
                             rows_v.at[b, j], gsem.at[b])

    def fire_scatters(b, islot, brow):
        for j in range(LGRP):
            pltpu.async_copy(rows_v.at[b, j],
                             agg_sh.at[didx_v.at[islot, brow + j]],
                             ssem.at[b], add=True)

    stage(0, 0)
    plsc.subcore_barrier()

    def outer(i, _):
        for v in range(UNROLL):
            g = i * UNROLL + v
            b = v % 4
            islot_a = (v // BLKG) % 2
            brow_a = (v % BLKG) * LGRP
            if v % BLKG == 0:
                drain_idx(islot_a)

            @pl.when(g >= 4)
            def _():
                drain_rows(b, ssem)

            fire_gathers(b, islot_a, brow_a)
            if v % BLKG == 3:
                blk = g // BLKG + 1

                @pl.when(blk < NBLK)
                def _():
                    stage(blk, (islot_a + 1) % 2)

            bp = (v - 2) % 4
            vb = (v - 2) % UNROLL
            islot_b = (vb // BLKG) % 2
            brow_b = (vb % BLKG) * LGRP

            @pl.when(g >= 2)
            def _():
                drain_rows(bp, gsem)
                fire_scatters(bp, islot_b, brow_b)

        return 0

    lax.fori_loop(0, LGROUPS // UNROLL, outer, 0)
    for gp in (LGROUPS - 2, LGROUPS - 1):
        v = gp % UNROLL
        drain_rows(v % 4, gsem)
        fire_scatters(v % 4, (v // BLKG) % 2, (v % BLKG) * LGRP)
    for b in range(4):
        drain_rows(b, ssem)
    plsc.subcore_barrier()
    pltpu.sync_copy(agg_sh.at[pl.ds(t * TILE_NODES, TILE_NODES)],
                    s_hbm.at[c, pl.ds(t * TILE_NODES, TILE_NODES)])


def _sc_layer(uflat, src2, dst2d, z16, drows):
    return pl.kernel(
        _layer_body,
        out_type=jax.ShapeDtypeStruct((NC, NPAD, HALF), jnp.float32),
        mesh=_mesh,
        compiler_params=_sc_params,
        scratch_types=[
            pltpu.VMEM((2, BROWS, 128), jnp.int32),
            pltpu.VMEM((2, BROWS, 128), jnp.int32),
            pltpu.VMEM((4, LGRP, 128, HALF), jnp.float32),
            pltpu.VMEM_SHARED((NPAD, HALF), jnp.float32),
            pltpu.SemaphoreType.DMA((4,)),
            pltpu.SemaphoreType.DMA((4,)),
            pltpu.SemaphoreType.DMA((2,)),
        ],
    )(uflat, src2, dst2d, z16, drows)


def _pool_body(h_hbm, batch_hbm, zp_hbm, out_hbm, bidx_v, rows2_v, pool_sh,
               psem):
    c = lax.axis_index("c")
    t = lax.axis_index("s")

    @pl.when(t == 0)
    def _():
        pltpu.sync_copy(zp_hbm, pool_sh)

    XCH = TILE_NODES // 128
    pltpu.sync_copy(batch_hbm.at[pl.ds(t * XCH, XCH)], bidx_v)
    plsc.subcore_barrier()
    base = t * TILE_NODES

    def fire(ch, slot):
        pltpu.async_copy(h_hbm.at[pl.ds(c * NPAD + base + ch * 128, 128)],
                         rows2_v.at[slot], psem.at[slot])

    fire(0, 0)

    def chunk2(i, _):
        for b in range(2):
            ch = i * 2 + b

            @pl.when(ch + 1 < XCH)
            def _():
                fire(ch + 1, (b + 1) % 2)

            pltpu.make_async_copy(h_hbm.at[pl.ds(0, 128)], rows2_v.at[b],
                                  psem.at[b]).wait()
            pltpu.sync_copy(rows2_v.at[b], pool_sh.at[bidx_v.at[ch]],
                            add=True)
        return 0

    lax.fori_loop(0, XCH // 2, chunk2, 0)
    plsc.subcore_barrier()

    @pl.when(t == 0)
    def _():
        pltpu.sync_copy(pool_sh, out_hbm.at[c])


def _sc_pool(hflat, batch_pad, zpool):
    return pl.kernel(
        _pool_body,
        out_type=jax.ShapeDtypeStruct((NC, NGPAD, HALF), jnp.float32),
        mesh=_mesh,
        compiler_params=_sc_params,
        scratch_types=[
            pltpu.VMEM((TILE_NODES // 128, 128), jnp.int32),
            pltpu.VMEM((2, 128, HALF), jnp.float32),
            pltpu.VMEM_SHARED((NGPAD, HALF), jnp.float32),
            pltpu.SemaphoreType.DMA((2,)),
        ],
    )(hflat, batch_pad, zpool)


def _table_kernel(emb_ref, pw0_ref, pb0_ref, pw1_ref, pb1_ref, cw0_ref,
                  out_ref):
    tab = jnp.maximum(emb_ref[...] @ pw0_ref[...] + pb0_ref[...], 0.0)
    tab = jnp.maximum(tab @ pw1_ref[...] + pb1_ref[...], 0.0)
    tab = tab @ cw0_ref[...]
    out_ref[...] = jnp.concatenate([tab[:, :HALF], tab[:, HALF:]], axis=0)


def _tc_table(emb, pw0, pb0, pw1, pb1, cw0):
    return pl.pallas_call(
        _table_kernel,
        out_shape=jax.ShapeDtypeStruct((2 * ENC, HALF), jnp.float32),
    )(emb, pw0, pb0, pw1, pb1, cw0)


def _xprep_kernel(x_ref, out_ref):
    xx = x_ref[...]
    out_ref[...] = jnp.stack([xx, xx + ENC], axis=0)


def _tc_xprep(x2d):
    return pl.pallas_call(
        _xprep_kernel,
        out_shape=jax.ShapeDtypeStruct((2, NPAD // 128, 128), jnp.int32),
    )(x2d)


def _prep_kernel(src_ref, out_ref):
    s = src_ref[...]
    out_ref[...] = jnp.stack([s, s + NPAD], axis=0)


def _tc_prep(src2d):
    return pl.pallas_call(
        _prep_kernel,
        grid=(112,),
        in_specs=[pl.BlockSpec((112, 128), lambda i: (i, 0))],
        out_specs=pl.BlockSpec((2, 112, 128), lambda i: (0, i, 0)),
        out_shape=jax.ShapeDtypeStruct((2, EROWS, 128), jnp.int32),
    )(src2d)


def _init_kernel(deg_ref, t1x_ref, dinv_ref, u0_ref):
    deg = deg_ref[0] + deg_ref[1] + 1.0
    dinv = lax.rsqrt(jnp.maximum(deg, 1.0))
    dinv_ref[...] = dinv
    u0_ref[...] = t1x_ref[...] * dinv[None]


def _tc_init(deg, t1x):
    R = NR // 32
    return pl.pallas_call(
        _init_kernel,
        grid=(32,),
        in_specs=[
            pl.BlockSpec((2, R, 128), lambda i: (0, i, 0)),
            pl.BlockSpec((2, R, 128), lambda i: (0, i, 0)),
        ],
        out_specs=[
            pl.BlockSpec((R, 128), lambda i: (i, 0)),
            pl.BlockSpec((2, R, 128), lambda i: (0, i, 0)),
        ],
        out_shape=[
            jax.ShapeDtypeStruct((NR, 128), jnp.float32),
            jax.ShapeDtypeStruct((2, NR, 128), jnp.float32),
        ],
    )(deg, t1x)


def _mid_kernel(s_ref, u_ref, dinv_ref, bl_ref, bh_ref,
                kaa_ref, kba_ref, kab_ref, kbb_ref, out_ref):
    dinv = dinv_ref[...]
    hl = jnp.maximum(dinv * (s_ref[0] + u_ref[0]) + bl_ref[...], 0.0)
    hh = jnp.maximum(dinv * (s_ref[1] + u_ref[1]) + bh_ref[...], 0.0)
    ul = jnp.dot(hl, kaa_ref[...], preferred_element_type=jnp.float32)
    ul += jnp.dot(hh, kba_ref[...], preferred_element_type=jnp.float32)
    uh = jnp.dot(hl, kab_ref[...], preferred_element_type=jnp.float32)
    uh += jnp.dot(hh, kbb_ref[...], preferred_element_type=jnp.float32)
    out_ref[...] = jnp.stack([ul * dinv, uh * dinv], axis=0)


def _last_kernel(s_ref, u_ref, dinv_ref, bl_ref, bh_ref, out_ref):
    dinv = dinv_ref[...]
    hl = jnp.maximum(dinv * (s_ref[0] + u_ref[0]) + bl_ref[...], 0.0)
    hh = jnp.maximum(dinv * (s_ref[1] + u_ref[1]) + bh_ref[...], 0.0)
    out_ref[...] = jnp.stack([hl, hh], axis=0)


def _tc_mid(s, u, dinv, bl, bh, ks):
    R = NR // 32
    return pl.pallas_call(
        _mid_kernel,
        grid=(32,),
        in_specs=[
            pl.BlockSpec((2, R, 128), lambda i: (0, i, 0)),
            pl.BlockSpec((2, R, 128), lambda i: (0, i, 0)),
            pl.BlockSpec((R, 128), lambda i: (i, 0)),
            pl.BlockSpec((1, 128), lambda i: (0, 0)),
            pl.BlockSpec((1, 128), lambda i: (0, 0)),
            pl.BlockSpec((128, 128), lambda i: (0, 0)),
            pl.BlockSpec((128, 128), lambda i: (0, 0)),
            pl.BlockSpec((128, 128), lambda i: (0, 0)),
            pl.BlockSpec((128, 128), lambda i: (0, 0)),
        ],
        out_specs=pl.BlockSpec((2, R, 128), lambda i: (0, i, 0)),
        out_shape=jax.ShapeDtypeStruct((2, NR, 128), jnp.float32),
    )(s, u, dinv, bl, bh, *ks)


def _tc_last(s, u, dinv, bl, bh):
    R = NR // 32
    return pl.pallas_call(
        _last_kernel,
        grid=(32,),
        in_specs=[
            pl.BlockSpec((2, R, 128), lambda i: (0, i, 0)),
            pl.BlockSpec((2, R, 128), lambda i: (0, i, 0)),
            pl.BlockSpec((R, 128), lambda i: (i, 0)),
            pl.BlockSpec((1, 128), lambda i: (0, 0)),
            pl.BlockSpec((1, 128), lambda i: (0, 0)),
        ],
        out_specs=pl.BlockSpec((2, R, 128), lambda i: (0, i, 0)),
        out_shape=jax.ShapeDtypeStruct((2, NR, 128), jnp.float32),
    )(s, u, dinv, bl, bh)


def _post_kernel(p_ref, w0_ref, b0_ref, w1_ref, b1_ref, pw_ref, pb_ref,
                 out_ref):
    g = jnp.concatenate([p_ref[0, :NG], p_ref[1, :NG]], axis=-1)
    g = jnp.maximum(g @ w0_ref[...] + b0_ref[...], 0.0)
    g = jnp.maximum(g @ w1_ref[...] + b1_ref[...], 0.0)
    props = g @ pw_ref[...] + pb_ref[...]
    out_ref[...] = props[:, 0:1]


def _tc_post(pooled, w0, b0, w1, b1, pw, pb):
    return pl.pallas_call(
        _post_kernel,
        out_shape=jax.ShapeDtypeStruct((NG, 1), jnp.float32),
    )(pooled, w0, b0, w1, b1, pw, pb)


def kernel(x, edge_index, batch, emb, pre_W, pre_b, conv_W, conv_b,
           post_W, post_b, prop_W, prop_b):
    f32 = jnp.float32
    x_pad = jnp.concatenate([x, jnp.zeros((NPAD - N,), jnp.int32)])
    batch_pad = jnp.concatenate(
        [batch, jnp.full((NPAD - N,), NG, jnp.int32)])
    pad_e = EPAD - E
    src_pad = jnp.concatenate(
        [edge_index[0],
         (jnp.arange(pad_e, dtype=jnp.int32) * 997) % N])
    dst_pad = jnp.concatenate(
        [edge_index[1], jnp.full((pad_e,), N, jnp.int32)])
    src2d = src_pad.reshape(EROWS, 128)
    dst2d = dst_pad.reshape(EROWS, 128)
    ones16 = jnp.ones((128, HALF), f32)
    z16 = jnp.zeros((TILE_NODES, HALF), f32)
    drows = jnp.zeros((LGRP, 128, HALF), f32)
    zpool = jnp.zeros((NGPAD, HALF), f32)
    eye8 = jnp.eye(8, dtype=f32)
    ks = [[jnp.kron(eye8, w[:HALF, :HALF]), jnp.kron(eye8, w[HALF:, :HALF]),
           jnp.kron(eye8, w[:HALF, HALF:]), jnp.kron(eye8, w[HALF:, HALF:])]
          for w in conv_W[1:]]
    bl = [jnp.tile(b[:HALF], 8).reshape(1, 128) for b in conv_b]
    bh = [jnp.tile(b[HALF:], 8).reshape(1, 128) for b in conv_b]
    pb2 = [b.reshape(1, CH) for b in pre_b]

    tab = _tc_table(emb, pre_W[0], pb2[0], pre_W[1], pb2[1], conv_W[0])
    xoff = _tc_xprep(x_pad.reshape(NPAD // 128, 128))
    xoff = xoff.reshape(2, NPAD // 128, 128)
    src2 = _tc_prep(src2d)
    deg, t1x = _sc_degree(dst2d, xoff, tab, z16, ones16)
    dinv, u = _tc_init(deg.reshape(2, NR, 128), t1x.reshape(2, NR, 128))
    h2 = None
    for l in range(6):
        s = _sc_layer(u.reshape(2 * NPAD, HALF), src2, dst2d, z16, drows)
        spk = s.reshape(2, NR, 128)
        if l < 5:
            u = _tc_mid(spk, u, dinv, bl[l], bh[l], ks[l])
        else:
            h2 = _tc_last(spk, u, dinv, bl[l], bh[l])
    pooled = _sc_pool(h2.reshape(2 * NPAD, HALF),
                      batch_pad.reshape(NPAD // 128, 128), zpool)
    out = _tc_post(pooled, post_W[0], post_b[0].reshape(1, -1),
                   post_W[1], post_b[1].reshape(1, -1),
                   prop_W, prop_b.reshape(1, -1))
    return out


# speedup vs baseline: 49.0683x; 1.0291x over previous
import jax
import jax.numpy as jnp
from jax import lax
from jax.experimental import pallas as pl
from jax.experimental.pallas import tpu as pltpu
from jax.experimental.pallas import tpu_sc as plsc

N = 100000
NPAD = 102400
E = 1600000
EPAD = 1605632
EROWS = 12544
CH = 32
HALF = 16
ENC = 128
NG = 256
NGPAD = 264
NC = 2
NS = 16
NR = NPAD // 8
TILE_NODES = NPAD // NS
LROWS = EROWS // NS
DROWS = EROWS // (NC * NS)
DGRP = 7

_mesh = plsc.VectorSubcoreMesh(core_axis_name="c", subcore_axis_name="s",
                               num_cores=NC, num_subcores=NS)
_sc_params = pltpu.CompilerParams(use_tc_tiling_on_sc=False)


def _deg_body(dst_hbm, xoff_hbm, tab_hbm, z16_hbm, ones16_hbm, deg_hbm,
              t1x_hbm, didx_v, ones_v, xi_v, rows2_v, deg_sh, asem, dsem,
              ssem):
    c = lax.axis_index("c")
    t = lax.axis_index("s")
    pltpu.sync_copy(z16_hbm, deg_sh.at[pl.ds(t * TILE_NODES, TILE_NODES)])
    pltpu.sync_copy(ones16_hbm, ones_v)
    XCH = TILE_NODES // 128
    pltpu.sync_copy(xoff_hbm.at[c, pl.ds(t * XCH, XCH)], xi_v)

    def fire(ch, slot):
        pltpu.async_copy(tab_hbm.at[xi_v.at[ch]], rows2_v.at[slot],
                         asem.at[slot])

    fire(0, 0)

    def chunk2(i, _):
        for b in range(2):
            ch = i * 2 + b

            @pl.when(ch + 1 < XCH)
            def _():
                fire(ch + 1, (b + 1) % 2)

            pltpu.make_async_copy(tab_hbm.at[xi_v.at[0]], rows2_v.at[b],
                                  asem.at[b]).wait()
            off = t * TILE_NODES + ch * 128
            pltpu.sync_copy(rows2_v.at[b], t1x_hbm.at[c, pl.ds(off, 128)])
        return 0

    lax.fori_loop(0, XCH // 2, chunk2, 0)

    base = (c * NS + t) * DROWS
    NDG = DROWS // DGRP

    def stage_d(g, slot):
        pltpu.async_copy(dst_hbm.at[pl.ds(base + g * DGRP, DGRP)],
                         didx_v.at[slot], dsem.at[slot])

    def drain_didx(slot):
        pltpu.make_async_copy(dst_hbm.at[pl.ds(base, DGRP)],
                              didx_v.at[slot], dsem.at[slot]).wait()

    def drain_scat(slot):
        for j in range(DGRP):
            pltpu.make_async_copy(ones16_hbm, rows2_v.at[0],
                                  ssem.at[slot]).wait()

    stage_d(0, 0)
    stage_d(1, 1)

    def grp4(i, _):
        for b in range(4):
            g = i * 4 + b

            @pl.when(g >= 2)
            def _():
                drain_scat((b + 2) % 4)

            drain_didx(b)
            for j in range(DGRP):
                pltpu.async_copy(ones_v, deg_sh.at[didx_v.at[b, j]],
                                 ssem.at[b], add=True)

            @pl.when(g + 2 < NDG)
            def _():
                stage_d(g + 2, (b + 2) % 4)
        return 0

    lax.fori_loop(0, NDG // 4, grp4, 0)
    for g in (NDG - 2, NDG - 1):
        drain_scat(g % 4)
    plsc.subcore_barrier()
    pltpu.sync_copy(deg_sh.at[pl.ds(t * TILE_NODES, TILE_NODES)],
                    deg_hbm.at[c, pl.ds(t * TILE_NODES, TILE_NODES)])


def _sc_degree(dst2d, xoff, tab, z16, ones16):
    return pl.kernel(
        _deg_body,
        out_type=[
            jax.ShapeDtypeStruct((NC, NPAD, HALF), jnp.float32),
            jax.ShapeDtypeStruct((NC, NPAD, HALF), jnp.float32),
        ],
        mesh=_mesh,
        compiler_params=_sc_params,
        scratch_types=[
            pltpu.VMEM((4, DGRP, 128), jnp.int32),
            pltpu.VMEM((128, HALF), jnp.float32),
            pltpu.VMEM((TILE_NODES // 128, 128), jnp.int32),
            pltpu.VMEM((2, 128, HALF), jnp.float32),
            pltpu.VMEM_SHARED((NPAD, HALF), jnp.float32),
            pltpu.SemaphoreType.DMA((2,)),
            pltpu.SemaphoreType.DMA((4,)),
            pltpu.SemaphoreType.DMA((4,)),
        ],
    )(dst2d, xoff, tab, z16, ones16)


LGRP = 2
LGROUPS = LROWS // LGRP
BLKG = 4
BROWS = BLKG * LGRP
NBLK = LGROUPS // BLKG
UNROLL = 8


def _layer_body(u_hbm, src2_hbm, dst_hbm, z16_hbm, drows_hbm, s_hbm,
                idx_v, didx_v, rows_v, agg_sh, gsem, ssem, isem):
    c = lax.axis_index("c")
    t = lax.axis_index("s")
    pltpu.sync_copy(z16_hbm, agg_sh.at[pl.ds(t * TILE_NODES, TILE_NODES)])
    base = t * LROWS

    def stage(k, islot):
        r0 = base + k * BROWS
        pltpu.async_copy(src2_hbm.at[c, pl.ds(r0, BROWS)], idx_v.at[islot],
                         isem.at[islot])
        pltpu.async_copy(dst_hbm.at[pl.ds(r0, BROWS)], didx_v.at[islot],
                         isem.at[islot])

    def drain_idx(islot):
        pltpu.make_async_copy(src2_hbm.at[c, pl.ds(base, BROWS)],
                              idx_v.at[islot], isem.at[islot]).wait()
        pltpu.make_async_copy(dst_hbm.at[pl.ds(base, BROWS)],
                              didx_v.at[islot], isem.at[islot]).wait()

    def drain_rows(b, sem):
        pltpu.make_async_copy(drows_hbm, rows_v.at[b], sem.at[b]).wait()

    def fire_gathers(b, islot, brow):
        for j in range(LGRP):
            pltpu.async_copy(u_hbm.at[idx_v.at[islot, brow + j]],
                             rows_v.at[b, j], gsem.at[b])

    def fire_scatters(b, islot, brow):
        for j in range(LGRP):
            pltpu.async_copy(rows_v.at[b, j],
                             agg_sh.at[didx_v.at[islot, brow + j]],
                             ssem.at[b], add=True)

    stage(0, 0)
    plsc.subcore_barrier()

    def outer(i, _):
        for v in range(UNROLL):
            g = i * UNROLL + v
            b = v % 4
            islot_a = (v // BLKG) % 2
            brow_a = (v % BLKG) * LGRP
            if v % BLKG == 0:
                drain_idx(islot_a)

            @pl.when(g >= 4)
            def _():
                drain_rows(b, ssem)

            fire_gathers(b, islot_a, brow_a)
            if v % BLKG == 3:
                blk = g // BLKG + 1

                @pl.when(blk < NBLK)
                def _():
                    stage(blk, (islot_a + 1) % 2)

            bp = (v - 2) % 4
            vb = (v - 2) % UNROLL
            islot_b = (vb // BLKG) % 2
            brow_b = (vb % BLKG) * LGRP

            @pl.when(g >= 2)
            def _():
                drain_rows(bp, gsem)
                fire_scatters(bp, islot_b, brow_b)

        return 0

    lax.fori_loop(0, LGROUPS // UNROLL, outer, 0)
    for gp in (LGROUPS - 2, LGROUPS - 1):
        v = gp % UNROLL
        drain_rows(v % 4, gsem)
        fire_scatters(v % 4, (v // BLKG) % 2, (v % BLKG) * LGRP)
    for b in range(4):
        drain_rows(b, ssem)
    plsc.subcore_barrier()
    pltpu.sync_copy(agg_sh.at[pl.ds(t * TILE_NODES, TILE_NODES)],
                    s_hbm.at[c, pl.ds(t * TILE_NODES, TILE_NODES)])


def _sc_layer(uflat, src2, dst2d, z16, drows):
    return pl.kernel(
        _layer_body,
        out_type=jax.ShapeDtypeStruct((NC, NPAD, HALF), jnp.float32),
        mesh=_mesh,
        compiler_params=_sc_params,
        scratch_types=[
            pltpu.VMEM((2, BROWS, 128), jnp.int32),
            pltpu.VMEM((2, BROWS, 128), jnp.int32),
            pltpu.VMEM((4, LGRP, 128, HALF), jnp.float32),
            pltpu.VMEM_SHARED((NPAD, HALF), jnp.float32),
            pltpu.SemaphoreType.DMA((4,)),
            pltpu.SemaphoreType.DMA((4,)),
            pltpu.SemaphoreType.DMA((2,)),
        ],
    )(uflat, src2, dst2d, z16, drows)


def _pool_body(h_hbm, batch_hbm, zp_hbm, out_hbm, bidx_v, rows2_v, pool_sh,
               psem):
    c = lax.axis_index("c")
    t = lax.axis_index("s")

    @pl.when(t == 0)
    def _():
        pltpu.sync_copy(zp_hbm, pool_sh)

    XCH = TILE_NODES // 128
    pltpu.sync_copy(batch_hbm.at[pl.ds(t * XCH, XCH)], bidx_v)
    plsc.subcore_barrier()
    base = t * TILE_NODES

    def fire(ch, slot):
        pltpu.async_copy(h_hbm.at[pl.ds(c * NPAD + base + ch * 128, 128)],
                         rows2_v.at[slot], psem.at[slot])

    fire(0, 0)

    def chunk2(i, _):
        for b in range(2):
            ch = i * 2 + b

            @pl.when(ch + 1 < XCH)
            def _():
                fire(ch + 1, (b + 1) % 2)

            pltpu.make_async_copy(h_hbm.at[pl.ds(0, 128)], rows2_v.at[b],
                                  psem.at[b]).wait()
            pltpu.sync_copy(rows2_v.at[b], pool_sh.at[bidx_v.at[ch]],
                            add=True)
        return 0

    lax.fori_loop(0, XCH // 2, chunk2, 0)
    plsc.subcore_barrier()

    @pl.when(t == 0)
    def _():
        pltpu.sync_copy(pool_sh, out_hbm.at[c])


def _sc_pool(hflat, batch_pad, zpool):
    return pl.kernel(
        _pool_body,
        out_type=jax.ShapeDtypeStruct((NC, NGPAD, HALF), jnp.float32),
        mesh=_mesh,
        compiler_params=_sc_params,
        scratch_types=[
            pltpu.VMEM((TILE_NODES // 128, 128), jnp.int32),
            pltpu.VMEM((2, 128, HALF), jnp.float32),
            pltpu.VMEM_SHARED((NGPAD, HALF), jnp.float32),
            pltpu.SemaphoreType.DMA((2,)),
        ],
    )(hflat, batch_pad, zpool)


def _table_kernel(emb_ref, pw0_ref, pb0_ref, pw1_ref, pb1_ref, cw0_ref,
                  out_ref):
    tab = jnp.maximum(emb_ref[...] @ pw0_ref[...] + pb0_ref[...], 0.0)
    tab = jnp.maximum(tab @ pw1_ref[...] + pb1_ref[...], 0.0)
    tab = tab @ cw0_ref[...]
    out_ref[...] = jnp.concatenate([tab[:, :HALF], tab[:, HALF:]], axis=0)


def _tc_table(emb, pw0, pb0, pw1, pb1, cw0):
    return pl.pallas_call(
        _table_kernel,
        out_shape=jax.ShapeDtypeStruct((2 * ENC, HALF), jnp.float32),
    )(emb, pw0, pb0, pw1, pb1, cw0)


def _xprep_kernel(x_ref, out_ref):
    xx = x_ref[...]
    out_ref[...] = jnp.stack([xx, xx + ENC], axis=0)


def _tc_xprep(x2d):
    return pl.pallas_call(
        _xprep_kernel,
        out_shape=jax.ShapeDtypeStruct((2, NPAD // 128, 128), jnp.int32),
    )(x2d)


def _prep_kernel(src_ref, out_ref):
    s = src_ref[...]
    out_ref[...] = jnp.stack([s, s + NPAD], axis=0)


def _tc_prep(src2d):
    return pl.pallas_call(
        _prep_kernel,
        grid=(16,),
        in_specs=[pl.BlockSpec((784, 128), lambda i: (i, 0))],
        out_specs=pl.BlockSpec((2, 784, 128), lambda i: (0, i, 0)),
        out_shape=jax.ShapeDtypeStruct((2, EROWS, 128), jnp.int32),
    )(src2d)


def _init_kernel(deg_ref, t1x_ref, dinv_ref, u0_ref):
    deg = deg_ref[0] + deg_ref[1] + 1.0
    dinv = lax.rsqrt(jnp.maximum(deg, 1.0))
    dinv_ref[...] = dinv
    u0_ref[...] = t1x_ref[...] * dinv[None]


def _tc_init(deg, t1x):
    R = NR // 32
    return pl.pallas_call(
        _init_kernel,
        grid=(32,),
        in_specs=[
            pl.BlockSpec((2, R, 128), lambda i: (0, i, 0)),
            pl.BlockSpec((2, R, 128), lambda i: (0, i, 0)),
        ],
        out_specs=[
            pl.BlockSpec((R, 128), lambda i: (i, 0)),
            pl.BlockSpec((2, R, 128), lambda i: (0, i, 0)),
        ],
        out_shape=[
            jax.ShapeDtypeStruct((NR, 128), jnp.float32),
            jax.ShapeDtypeStruct((2, NR, 128), jnp.float32),
        ],
    )(deg, t1x)


def _mid_kernel(s_ref, u_ref, dinv_ref, bl_ref, bh_ref,
                kaa_ref, kba_ref, kab_ref, kbb_ref, out_ref):
    dinv = dinv_ref[...]
    hl = jnp.maximum(dinv * (s_ref[0] + u_ref[0]) + bl_ref[...], 0.0)
    hh = jnp.maximum(dinv * (s_ref[1] + u_ref[1]) + bh_ref[...], 0.0)
    ul = jnp.dot(hl, kaa_ref[...], preferred_element_type=jnp.float32)
    ul += jnp.dot(hh, kba_ref[...], preferred_element_type=jnp.float32)
    uh = jnp.dot(hl, kab_ref[...], preferred_element_type=jnp.float32)
    uh += jnp.dot(hh, kbb_ref[...], preferred_element_type=jnp.float32)
    out_ref[...] = jnp.stack([ul * dinv, uh * dinv], axis=0)


def _last_kernel(s_ref, u_ref, dinv_ref, bl_ref, bh_ref, out_ref):
    dinv = dinv_ref[...]
    hl = jnp.maximum(dinv * (s_ref[0] + u_ref[0]) + bl_ref[...], 0.0)
    hh = jnp.maximum(dinv * (s_ref[1] + u_ref[1]) + bh_ref[...], 0.0)
    out_ref[...] = jnp.stack([hl, hh], axis=0)


def _tc_mid(s, u, dinv, bl, bh, ks):
    R = NR // 32
    return pl.pallas_call(
        _mid_kernel,
        grid=(32,),
        in_specs=[
            pl.BlockSpec((2, R, 128), lambda i: (0, i, 0)),
            pl.BlockSpec((2, R, 128), lambda i: (0, i, 0)),
            pl.BlockSpec((R, 128), lambda i: (i, 0)),
            pl.BlockSpec((1, 128), lambda i: (0, 0)),
            pl.BlockSpec((1, 128), lambda i: (0, 0)),
            pl.BlockSpec((128, 128), lambda i: (0, 0)),
            pl.BlockSpec((128, 128), lambda i: (0, 0)),
            pl.BlockSpec((128, 128), lambda i: (0, 0)),
            pl.BlockSpec((128, 128), lambda i: (0, 0)),
        ],
        out_specs=pl.BlockSpec((2, R, 128), lambda i: (0, i, 0)),
        out_shape=jax.ShapeDtypeStruct((2, NR, 128), jnp.float32),
    )(s, u, dinv, bl, bh, *ks)


def _tc_last(s, u, dinv, bl, bh):
    R = NR // 32
    return pl.pallas_call(
        _last_kernel,
        grid=(32,),
        in_specs=[
            pl.BlockSpec((2, R, 128), lambda i: (0, i, 0)),
            pl.BlockSpec((2, R, 128), lambda i: (0, i, 0)),
            pl.BlockSpec((R, 128), lambda i: (i, 0)),
            pl.BlockSpec((1, 128), lambda i: (0, 0)),
            pl.BlockSpec((1, 128), lambda i: (0, 0)),
        ],
        out_specs=pl.BlockSpec((2, R, 128), lambda i: (0, i, 0)),
        out_shape=jax.ShapeDtypeStruct((2, NR, 128), jnp.float32),
    )(s, u, dinv, bl, bh)


def _post_kernel(p_ref, w0_ref, b0_ref, w1_ref, b1_ref, pw_ref, pb_ref,
                 out_ref):
    g = jnp.concatenate([p_ref[0, :NG], p_ref[1, :NG]], axis=-1)
    g = jnp.maximum(g @ w0_ref[...] + b0_ref[...], 0.0)
    g = jnp.maximum(g @ w1_ref[...] + b1_ref[...], 0.0)
    props = g @ pw_ref[...] + pb_ref[...]
    out_ref[...] = props[:, 0:1]


def _tc_post(pooled, w0, b0, w1, b1, pw, pb):
    return pl.pallas_call(
        _post_kernel,
        out_shape=jax.ShapeDtypeStruct((NG, 1), jnp.float32),
    )(pooled, w0, b0, w1, b1, pw, pb)


def kernel(x, edge_index, batch, emb, pre_W, pre_b, conv_W, conv_b,
           post_W, post_b, prop_W, prop_b):
    f32 = jnp.float32
    x_pad = jnp.concatenate([x, jnp.zeros((NPAD - N,), jnp.int32)])
    batch_pad = jnp.concatenate(
        [batch, jnp.full((NPAD - N,), NG, jnp.int32)])
    pad_e = EPAD - E
    src_pad = jnp.concatenate(
        [edge_index[0],
         (jnp.arange(pad_e, dtype=jnp.int32) * 997) % N])
    dst_pad = jnp.concatenate(
        [edge_index[1], jnp.full((pad_e,), N, jnp.int32)])
    src2d = src_pad.reshape(EROWS, 128)
    dst2d = dst_pad.reshape(EROWS, 128)
    ones16 = jnp.ones((128, HALF), f32)
    z16 = jnp.zeros((TILE_NODES, HALF), f32)
    drows = jnp.zeros((LGRP, 128, HALF), f32)
    zpool = jnp.zeros((NGPAD, HALF), f32)
    eye8 = jnp.eye(8, dtype=f32)

    def _kron8(w16):
        return (eye8[:, None, :, None] * w16[None, :, None, :]).reshape(
            8 * HALF, 8 * HALF)

    ks = [[_kron8(w[:HALF, :HALF]), _kron8(w[HALF:, :HALF]),
           _kron8(w[:HALF, HALF:]), _kron8(w[HALF:, HALF:])]
          for w in conv_W[1:]]
    bl = [jnp.tile(b[:HALF], 8).reshape(1, 128) for b in conv_b]
    bh = [jnp.tile(b[HALF:], 8).reshape(1, 128) for b in conv_b]
    pb2 = [b.reshape(1, CH) for b in pre_b]

    tab = _tc_table(emb, pre_W[0], pb2[0], pre_W[1], pb2[1], conv_W[0])
    xoff = _tc_xprep(x_pad.reshape(NPAD // 128, 128))
    xoff = xoff.reshape(2, NPAD // 128, 128)
    src2 = _tc_prep(src2d)
    deg, t1x = _sc_degree(dst2d, xoff, tab, z16, ones16)
    dinv, u = _tc_init(deg.reshape(2, NR, 128), t1x.reshape(2, NR, 128))
    h2 = None
    for l in range(6):
        s = _sc_layer(u.reshape(2 * NPAD, HALF), src2, dst2d, z16, drows)
        spk = s.reshape(2, NR, 128)
        if l < 5:
            u = _tc_mid(spk, u, dinv, bl[l], bh[l], ks[l])
        else:
            h2 = _tc_last(spk, u, dinv, bl[l], bh[l])
    pooled = _sc_pool(h2.reshape(2 * NPAD, HALF),
                      batch_pad.reshape(NPAD // 128, 128), zpool)
    out = _tc_post(pooled, post_W[0], post_b[0].reshape(1, -1),
                   post_W[1], post_b[1].reshape(1, -1),
                   prop_W, prop_b.reshape(1, -1))
    return out


# speedup vs baseline: 50.5527x; 1.0303x over previous
import jax
import jax.numpy as jnp
from jax import lax
from jax.experimental import pallas as pl
from jax.experimental.pallas import tpu as pltpu
from jax.experimental.pallas import tpu_sc as plsc

N = 100000
NPAD = 102400
E = 1600000
EPAD = 1605632
EROWS = 12544
CH = 32
HALF = 16
ENC = 128
NG = 256
NGPAD = 264
NC = 2
NS = 16
NR = NPAD // 8
TILE_NODES = NPAD // NS
LROWS = EROWS // NS
DROWS = EROWS // (NC * NS)
DGRP = 7

_mesh = plsc.VectorSubcoreMesh(core_axis_name="c", subcore_axis_name="s",
                               num_cores=NC, num_subcores=NS)
_sc_params = pltpu.CompilerParams(use_tc_tiling_on_sc=False)


def _deg_body(dst_hbm, xoff_hbm, tab_hbm, z16_hbm, ones16_hbm, deg_hbm,
              t1x_hbm, didx_v, ones_v, xi_v, rows2_v, deg_sh, asem, dsem,
              ssem):
    c = lax.axis_index("c")
    t = lax.axis_index("s")
    pltpu.sync_copy(z16_hbm, deg_sh.at[pl.ds(t * TILE_NODES, TILE_NODES)])
    pltpu.sync_copy(ones16_hbm, ones_v)
    XCH = TILE_NODES // 128
    pltpu.sync_copy(xoff_hbm.at[c, pl.ds(t * XCH, XCH)], xi_v)

    def fire(ch, slot):
        pltpu.async_copy(tab_hbm.at[xi_v.at[ch]], rows2_v.at[slot],
                         asem.at[slot])

    fire(0, 0)

    def chunk2(i, _):
        for b in range(2):
            ch = i * 2 + b

            @pl.when(ch + 1 < XCH)
            def _():
                fire(ch + 1, (b + 1) % 2)

            pltpu.make_async_copy(tab_hbm.at[xi_v.at[0]], rows2_v.at[b],
                                  asem.at[b]).wait()
            off = t * TILE_NODES + ch * 128
            pltpu.sync_copy(rows2_v.at[b], t1x_hbm.at[c, pl.ds(off, 128)])
        return 0

    lax.fori_loop(0, XCH // 2, chunk2, 0)

    base = (c * NS + t) * DROWS
    NDG = DROWS // DGRP

    def stage_d(g, slot):
        pltpu.async_copy(dst_hbm.at[pl.ds(base + g * DGRP, DGRP)],
                         didx_v.at[slot], dsem.at[slot])

    def drain_didx(slot):
        pltpu.make_async_copy(dst_hbm.at[pl.ds(base, DGRP)],
                              didx_v.at[slot], dsem.at[slot]).wait()

    def drain_scat(slot):
        for j in range(DGRP):
            pltpu.make_async_copy(ones16_hbm, rows2_v.at[0],
                                  ssem.at[slot]).wait()

    stage_d(0, 0)
    stage_d(1, 1)

    def grp4(i, _):
        for b in range(4):
            g = i * 4 + b

            @pl.when(g >= 2)
            def _():
                drain_scat((b + 2) % 4)

            drain_didx(b)
            for j in range(DGRP):
                pltpu.async_copy(ones_v, deg_sh.at[didx_v.at[b, j]],
                                 ssem.at[b], add=True)

            @pl.when(g + 2 < NDG)
            def _():
                stage_d(g + 2, (b + 2) % 4)
        return 0

    lax.fori_loop(0, NDG // 4, grp4, 0)
    for g in (NDG - 2, NDG - 1):
        drain_scat(g % 4)
    plsc.subcore_barrier()
    pltpu.sync_copy(deg_sh.at[pl.ds(t * TILE_NODES, TILE_NODES)],
                    deg_hbm.at[c, pl.ds(t * TILE_NODES, TILE_NODES)])


def _sc_degree(dst2d, xoff, tab, z16, ones16):
    return pl.kernel(
        _deg_body,
        out_type=[
            jax.ShapeDtypeStruct((NC, NPAD, HALF), jnp.float32),
            jax.ShapeDtypeStruct((NC, NPAD, HALF), jnp.float32),
        ],
        mesh=_mesh,
        compiler_params=_sc_params,
        scratch_types=[
            pltpu.VMEM((4, DGRP, 128), jnp.int32),
            pltpu.VMEM((128, HALF), jnp.float32),
            pltpu.VMEM((TILE_NODES // 128, 128), jnp.int32),
            pltpu.VMEM((2, 128, HALF), jnp.float32),
            pltpu.VMEM_SHARED((NPAD, HALF), jnp.float32),
            pltpu.SemaphoreType.DMA((2,)),
            pltpu.SemaphoreType.DMA((4,)),
            pltpu.SemaphoreType.DMA((4,)),
        ],
    )(dst2d, xoff, tab, z16, ones16)


LGRP = 2
LGROUPS = LROWS // LGRP
BLKG = 4
BROWS = BLKG * LGRP
NBLK = LGROUPS // BLKG
UNROLL = 8


def _layer_body(u_hbm, src2_hbm, dst_hbm, z16_hbm, drows_hbm, s_hbm,
                idx_v, didx_v, rows_v, agg_sh, gsem, ssem, isem):
    c = lax.axis_index("c")
    t = lax.axis_index("s")
    pltpu.sync_copy(z16_hbm, agg_sh.at[pl.ds(t * TILE_NODES, TILE_NODES)])
    base = t * LROWS

    def stage(k, islot):
        r0 = base + k * BROWS
        pltpu.async_copy(src2_hbm.at[c, pl.ds(r0, BROWS)], idx_v.at[islot],
                         isem.at[islot])
        pltpu.async_copy(dst_hbm.at[pl.ds(r0, BROWS)], didx_v.at[islot],
                         isem.at[islot])

    def drain_idx(islot):
        pltpu.make_async_copy(src2_hbm.at[c, pl.ds(base, BROWS)],
                              idx_v.at[islot], isem.at[islot]).wait()
        pltpu.make_async_copy(dst_hbm.at[pl.ds(base, BROWS)],
                              didx_v.at[islot], isem.at[islot]).wait()

    def drain_rows(b, sem):
        pltpu.make_async_copy(drows_hbm, rows_v.at[b], sem.at[b]).wait()

    def fire_gathers(b, islot, brow):
        for j in range(LGRP):
            pltpu.async_copy(u_hbm.at[idx_v.at[islot, brow + j]],
                             rows_v.at[b, j], gsem.at[b])

    def fire_scatters(b, islot, brow):
        for j in range(LGRP):
            pltpu.async_copy(rows_v.at[b, j],
                             agg_sh.at[didx_v.at[islot, brow + j]],
                             ssem.at[b], add=True)

    stage(0, 0)
    plsc.subcore_barrier()

    def outer(i, _):
        for v in range(UNROLL):
            g = i * UNROLL + v
            b = v % 4
            islot_a = (v // BLKG) % 2
            brow_a = (v % BLKG) * LGRP
            if v % BLKG == 0:
                drain_idx(islot_a)

            @pl.when(g >= 4)
            def _():
                drain_rows(b, ssem)

            fire_gathers(b, islot_a, brow_a)
            if v % BLKG == 3:
                blk = g // BLKG + 1

                @pl.when(blk < NBLK)
                def _():
                    stage(blk, (islot_a + 1) % 2)

            bp = (v - 2) % 4
            vb = (v - 2) % UNROLL
            islot_b = (vb // BLKG) % 2
            brow_b = (vb % BLKG) * LGRP

            @pl.when(g >= 2)
            def _():
                drain_rows(bp, gsem)
                fire_scatters(bp, islot_b, brow_b)

        return 0

    lax.fori_loop(0, LGROUPS // UNROLL, outer, 0)
    for gp in (LGROUPS - 2, LGROUPS - 1):
        v = gp % UNROLL
        drain_rows(v % 4, gsem)
        fire_scatters(v % 4, (v // BLKG) % 2, (v % BLKG) * LGRP)
    for b in range(4):
        drain_rows(b, ssem)
    plsc.subcore_barrier()
    pltpu.sync_copy(agg_sh.at[pl.ds(t * TILE_NODES, TILE_NODES)],
                    s_hbm.at[c, pl.ds(t * TILE_NODES, TILE_NODES)])


def _sc_layer(uflat, src2, dst2d, z16, drows):
    return pl.kernel(
        _layer_body,
        out_type=jax.ShapeDtypeStruct((NC, NPAD, HALF), jnp.float32),
        mesh=_mesh,
        compiler_params=_sc_params,
        scratch_types=[
            pltpu.VMEM((2, BROWS, 128), jnp.int32),
            pltpu.VMEM((2, BROWS, 128), jnp.int32),
            pltpu.VMEM((4, LGRP, 128, HALF), jnp.float32),
            pltpu.VMEM_SHARED((NPAD, HALF), jnp.float32),
            pltpu.SemaphoreType.DMA((4,)),
            pltpu.SemaphoreType.DMA((4,)),
            pltpu.SemaphoreType.DMA((2,)),
        ],
    )(uflat, src2, dst2d, z16, drows)


def _pool_body(h_hbm, batch_hbm, zp_hbm, out_hbm, bidx_v, rows2_v, pool_sh,
               psem):
    c = lax.axis_index("c")
    t = lax.axis_index("s")

    @pl.when(t == 0)
    def _():
        pltpu.sync_copy(zp_hbm, pool_sh)

    XCH = TILE_NODES // 128
    pltpu.sync_copy(batch_hbm.at[pl.ds(t * XCH, XCH)], bidx_v)
    plsc.subcore_barrier()
    base = t * TILE_NODES

    def fire(ch, slot):
        pltpu.async_copy(h_hbm.at[pl.ds(c * NPAD + base + ch * 128, 128)],
                         rows2_v.at[slot], psem.at[slot])

    fire(0, 0)

    def chunk2(i, _):
        for b in range(2):
            ch = i * 2 + b

            @pl.when(ch + 1 < XCH)
            def _():
                fire(ch + 1, (b + 1) % 2)

            pltpu.make_async_copy(h_hbm.at[pl.ds(0, 128)], rows2_v.at[b],
                                  psem.at[b]).wait()
            pltpu.sync_copy(rows2_v.at[b], pool_sh.at[bidx_v.at[ch]],
                            add=True)
        return 0

    lax.fori_loop(0, XCH // 2, chunk2, 0)
    plsc.subcore_barrier()

    @pl.when(t == 0)
    def _():
        pltpu.sync_copy(pool_sh, out_hbm.at[c])


def _sc_pool(hflat, batch_pad, zpool):
    return pl.kernel(
        _pool_body,
        out_type=jax.ShapeDtypeStruct((NC, NGPAD, HALF), jnp.float32),
        mesh=_mesh,
        compiler_params=_sc_params,
        scratch_types=[
            pltpu.VMEM((TILE_NODES // 128, 128), jnp.int32),
            pltpu.VMEM((2, 128, HALF), jnp.float32),
            pltpu.VMEM_SHARED((NGPAD, HALF), jnp.float32),
            pltpu.SemaphoreType.DMA((2,)),
        ],
    )(hflat, batch_pad, zpool)


def _table_kernel(emb_ref, pw0_ref, pb0_ref, pw1_ref, pb1_ref, cw0_ref,
                  out_ref):
    tab = jnp.maximum(emb_ref[...] @ pw0_ref[...] + pb0_ref[...], 0.0)
    tab = jnp.maximum(tab @ pw1_ref[...] + pb1_ref[...], 0.0)
    tab = tab @ cw0_ref[...]
    out_ref[...] = jnp.concatenate([tab[:, :HALF], tab[:, HALF:]], axis=0)


def _tc_table(emb, pw0, pb0, pw1, pb1, cw0):
    return pl.pallas_call(
        _table_kernel,
        out_shape=jax.ShapeDtypeStruct((2 * ENC, HALF), jnp.float32),
    )(emb, pw0, pb0, pw1, pb1, cw0)


def _xprep_kernel(x_ref, out_ref):
    xx = x_ref[...]
    out_ref[...] = jnp.stack([xx, xx + ENC], axis=0)


def _tc_xprep(x2d):
    return pl.pallas_call(
        _xprep_kernel,
        out_shape=jax.ShapeDtypeStruct((2, NPAD // 128, 128), jnp.int32),
    )(x2d)


def _prep_kernel(ei_ref, src2_ref, dst_ref):
    s = ei_ref[0]
    d = ei_ref[1]
    padi = lax.broadcasted_iota(jnp.int32, (EROWS - E // 128, 128), 1) % N
    sp = jnp.concatenate([s, padi], axis=0)
    dst_ref[...] = jnp.concatenate(
        [d, jnp.full((EROWS - E // 128, 128), N, jnp.int32)], axis=0)
    src2_ref[...] = jnp.stack([sp, sp + NPAD], axis=0)


def _tc_prep(ei3):
    return pl.pallas_call(
        _prep_kernel,
        out_shape=[
            jax.ShapeDtypeStruct((2, EROWS, 128), jnp.int32),
            jax.ShapeDtypeStruct((EROWS, 128), jnp.int32),
        ],
    )(ei3)


def _init_kernel(deg_ref, t1x_ref, dinv_ref, u0_ref):
    deg = deg_ref[0] + deg_ref[1] + 1.0
    dinv = lax.rsqrt(jnp.maximum(deg, 1.0))
    dinv_ref[...] = dinv
    u0_ref[...] = t1x_ref[...] * dinv[None]


def _tc_init(deg, t1x):
    R = NR // 32
    return pl.pallas_call(
        _init_kernel,
        grid=(32,),
        in_specs=[
            pl.BlockSpec((2, R, 128), lambda i: (0, i, 0)),
            pl.BlockSpec((2, R, 128), lambda i: (0, i, 0)),
        ],
        out_specs=[
            pl.BlockSpec((R, 128), lambda i: (i, 0)),
            pl.BlockSpec((2, R, 128), lambda i: (0, i, 0)),
        ],
        out_shape=[
            jax.ShapeDtypeStruct((NR, 128), jnp.float32),
            jax.ShapeDtypeStruct((2, NR, 128), jnp.float32),
        ],
    )(deg, t1x)


def _mid_kernel(s_ref, u_ref, dinv_ref, bl_ref, bh_ref,
                kaa_ref, kba_ref, kab_ref, kbb_ref, out_ref):
    dinv = dinv_ref[...]
    hl = jnp.maximum(dinv * (s_ref[0] + u_ref[0]) + bl_ref[...], 0.0)
    hh = jnp.maximum(dinv * (s_ref[1] + u_ref[1]) + bh_ref[...], 0.0)
    ul = jnp.dot(hl, kaa_ref[...], preferred_element_type=jnp.float32)
    ul += jnp.dot(hh, kba_ref[...], preferred_element_type=jnp.float32)
    uh = jnp.dot(hl, kab_ref[...], preferred_element_type=jnp.float32)
    uh += jnp.dot(hh, kbb_ref[...], preferred_element_type=jnp.float32)
    out_ref[...] = jnp.stack([ul * dinv, uh * dinv], axis=0)


def _last_kernel(s_ref, u_ref, dinv_ref, bl_ref, bh_ref, out_ref):
    dinv = dinv_ref[...]
    hl = jnp.maximum(dinv * (s_ref[0] + u_ref[0]) + bl_ref[...], 0.0)
    hh = jnp.maximum(dinv * (s_ref[1] + u_ref[1]) + bh_ref[...], 0.0)
    out_ref[...] = jnp.stack([hl, hh], axis=0)


def _tc_mid(s, u, dinv, bl, bh, ks):
    R = NR // 32
    return pl.pallas_call(
        _mid_kernel,
        grid=(32,),
        in_specs=[
            pl.BlockSpec((2, R, 128), lambda i: (0, i, 0)),
            pl.BlockSpec((2, R, 128), lambda i: (0, i, 0)),
            pl.BlockSpec((R, 128), lambda i: (i, 0)),
            pl.BlockSpec((1, 128), lambda i: (0, 0)),
            pl.BlockSpec((1, 128), lambda i: (0, 0)),
            pl.BlockSpec((128, 128), lambda i: (0, 0)),
            pl.BlockSpec((128, 128), lambda i: (0, 0)),
            pl.BlockSpec((128, 128), lambda i: (0, 0)),
            pl.BlockSpec((128, 128), lambda i: (0, 0)),
        ],
        out_specs=pl.BlockSpec((2, R, 128), lambda i: (0, i, 0)),
        out_shape=jax.ShapeDtypeStruct((2, NR, 128), jnp.float32),
    )(s, u, dinv, bl, bh, *ks)


def _tc_last(s, u, dinv, bl, bh):
    R = NR // 32
    return pl.pallas_call(
        _last_kernel,
        grid=(32,),
        in_specs=[
            pl.BlockSpec((2, R, 128), lambda i: (0, i, 0)),
            pl.BlockSpec((2, R, 128), lambda i: (0, i, 0)),
            pl.BlockSpec((R, 128), lambda i: (i, 0)),
            pl.BlockSpec((1, 128), lambda i: (0, 0)),
            pl.BlockSpec((1, 128), lambda i: (0, 0)),
        ],
        out_specs=pl.BlockSpec((2, R, 128), lambda i: (0, i, 0)),
        out_shape=jax.ShapeDtypeStruct((2, NR, 128), jnp.float32),
    )(s, u, dinv, bl, bh)


def _post_kernel(p_ref, w0_ref, b0_ref, w1_ref, b1_ref, pw_ref, pb_ref,
                 out_ref):
    g = jnp.concatenate([p_ref[0, :NG], p_ref[1, :NG]], axis=-1)
    g = jnp.maximum(g @ w0_ref[...] + b0_ref[...], 0.0)
    g = jnp.maximum(g @ w1_ref[...] + b1_ref[...], 0.0)
    props = g @ pw_ref[...] + pb_ref[...]
    out_ref[...] = props[:, 0:1]


def _tc_post(pooled, w0, b0, w1, b1, pw, pb):
    return pl.pallas_call(
        _post_kernel,
        out_shape=jax.ShapeDtypeStruct((NG, 1), jnp.float32),
    )(pooled, w0, b0, w1, b1, pw, pb)


def kernel(x, edge_index, batch, emb, pre_W, pre_b, conv_W, conv_b,
           post_W, post_b, prop_W, prop_b):
    f32 = jnp.float32
    x_pad = jnp.concatenate([x, jnp.zeros((NPAD - N,), jnp.int32)])
    batch_pad = jnp.concatenate(
        [batch, jnp.full((NPAD - N,), NG, jnp.int32)])
    ones16 = jnp.ones((128, HALF), f32)
    z16 = jnp.zeros((TILE_NODES, HALF), f32)
    drows = jnp.zeros((LGRP, 128, HALF), f32)
    zpool = jnp.zeros((NGPAD, HALF), f32)
    eye8 = jnp.eye(8, dtype=f32)

    def _kron8(w16):
        return (eye8[:, None, :, None] * w16[None, :, None, :]).reshape(
            8 * HALF, 8 * HALF)

    ks = [[_kron8(w[:HALF, :HALF]), _kron8(w[HALF:, :HALF]),
           _kron8(w[:HALF, HALF:]), _kron8(w[HALF:, HALF:])]
          for w in conv_W[1:]]
    bl = [jnp.tile(b[:HALF], 8).reshape(1, 128) for b in conv_b]
    bh = [jnp.tile(b[HALF:], 8).reshape(1, 128) for b in conv_b]
    pb2 = [b.reshape(1, CH) for b in pre_b]

    tab = _tc_table(emb, pre_W[0], pb2[0], pre_W[1], pb2[1], conv_W[0])
    xoff = _tc_xprep(x_pad.reshape(NPAD // 128, 128))
    xoff = xoff.reshape(2, NPAD // 128, 128)
    src2, dst2d = _tc_prep(edge_index.reshape(2, E // 128, 128))
    deg, t1x = _sc_degree(dst2d, xoff, tab, z16, ones16)
    dinv, u = _tc_init(deg.reshape(2, NR, 128), t1x.reshape(2, NR, 128))
    h2 = None
    for l in range(6):
        s = _sc_layer(u.reshape(2 * NPAD, HALF), src2, dst2d, z16, drows)
        spk = s.reshape(2, NR, 128)
        if l < 5:
            u = _tc_mid(spk, u, dinv, bl[l], bh[l], ks[l])
        else:
            h2 = _tc_last(spk, u, dinv, bl[l], bh[l])
    pooled = _sc_pool(h2.reshape(2 * NPAD, HALF),
                      batch_pad.reshape(NPAD // 128, 128), zpool)
    out = _tc_post(pooled, post_W[0], post_b[0].reshape(1, -1),
                   post_W[1], post_b[1].reshape(1, -1),
                   prop_W, prop_b.reshape(1, -1))
    return out
